# baseline (speedup 1.0000x reference)
import functools

import jax
import jax.numpy as jnp
from jax import lax
from jax.experimental import pallas as pl
from jax.experimental.pallas import tpu as pltpu
from jax.experimental.pallas import tpu_sc as plsc

N_T = 6000
N_O = 4000
N = N_T + N_O
NROWS = 10240
E = 320000
D_BASE = 128
D_OTHER = 256
EMB = 64
IN_DIM = D_BASE + EMB
HID = 64
OUT = 16
R = 4
CNT_STRIDE = 8

NC, NS = 2, 16
NW = NC * NS
CHUNK = 256
GROUP1 = 1
GROUP2 = 2
CHUNKS = 40
EPW = CHUNKS * CHUNK
EPAD = NW * EPW

_f32 = jnp.float32


def _a1_body(xo_ref, wmap_ref, bmap_ref, a0_ref, a1_ref, dst_ref, et_ref,
             ho_ref, aug_ref, cidx_ref):
    ho_ref[...] = (jnp.dot(xo_ref[...], wmap_ref[...],
                           preferred_element_type=_f32)
                   + bmap_ref[...][None, :])
    aug_ref[...] = (a0_ref[...] + a1_ref[...]) * 0.5
    cidx_ref[...] = dst_ref[...] * CNT_STRIDE + et_ref[...]


def _stage_a1(x_other, W_map, b_map, a0, a1, dst2d, et2d):
    return pl.pallas_call(
        _a1_body,
        out_shape=(
            jax.ShapeDtypeStruct((N_O, IN_DIM), _f32),
            jax.ShapeDtypeStruct((N_T, EMB), _f32),
            jax.ShapeDtypeStruct(dst2d.shape, jnp.int32),
        ),
    )(x_other, W_map, b_map, a0, a1, dst2d, et2d)


def _a2_body(h_ref, wi1_ref, wl1_ref, rel1_ref, wr1_ref, wi2_ref,
             g1_ref, l1_ref, rp1_ref, rp2_ref):
    h = h_ref[...]
    g1_ref[...] = jnp.dot(h, wi1_ref[...], preferred_element_type=_f32)
    l1_ref[...] = jnp.dot(h, wl1_ref[...], preferred_element_type=_f32)
    z = jnp.zeros((CNT_STRIDE - R, HID), _f32)
    rp1 = jnp.dot(rel1_ref[...], wi1_ref[...], preferred_element_type=_f32)
    rp1_ref[...] = jnp.concatenate([rp1, z], axis=0)
    rel2 = jnp.dot(rel1_ref[...], wr1_ref[...], preferred_element_type=_f32)
    rp2 = jnp.dot(rel2, wi2_ref[...], preferred_element_type=_f32)
    rp2_ref[...] = jnp.concatenate([rp2, jnp.zeros((CNT_STRIDE - R, OUT), _f32)],
                                   axis=0)


def _stage_a2(hp, W_in1, W_loop1, rel1, W_rel1, W_in2):
    nblk = 5
    rows = NROWS // nblk
    full = lambda s: pl.BlockSpec(s, lambda i: (0, 0))
    return pl.pallas_call(
        _a2_body,
        grid=(nblk,),
        in_specs=[
            pl.BlockSpec((rows, IN_DIM), lambda i: (i, 0)),
            full((IN_DIM, HID)), full((IN_DIM, HID)),
            full((R, IN_DIM)), full((IN_DIM, HID)), full((HID, OUT)),
        ],
        out_specs=(
            pl.BlockSpec((rows, HID), lambda i: (i, 0)),
            pl.BlockSpec((rows, HID), lambda i: (i, 0)),
            full((CNT_STRIDE, HID)),
            full((CNT_STRIDE, OUT)),
        ),
        out_shape=(
            jax.ShapeDtypeStruct((NROWS, HID), _f32),
            jax.ShapeDtypeStruct((NROWS, HID), _f32),
            jax.ShapeDtypeStruct((CNT_STRIDE, HID), _f32),
            jax.ShapeDtypeStruct((CNT_STRIDE, OUT), _f32),
        ),
    )(hp, W_in1, W_loop1, rel1, W_rel1, W_in2)


def _seg_kernel_body(width, group, phases, src_hbm, dst_hbm, cidx_hbm,
                     tab_hbm, zrow_hbm, zcnt_hbm, out_rows, out_cnt,
                     src_v, dst_v, cidx_v, buf0, buf1, ones_v, tab_sp,
                     acc, acc_cnt, sem_g0, sem_g1, sem_s0, sem_s1):
    cid = lax.axis_index("c")
    sid = lax.axis_index("s")
    wid = cid * NS + sid
    rpt = NROWS // NS
    pltpu.sync_copy(zrow_hbm.at[pl.ds(sid * rpt, rpt)],
                    acc.at[pl.ds(sid * rpt, rpt)])
    pltpu.sync_copy(tab_hbm.at[pl.ds(sid * rpt, rpt)],
                    tab_sp.at[pl.ds(sid * rpt, rpt)])
    do_cnt = cidx_hbm is not None
    if do_cnt:
        cpt = (NROWS * CNT_STRIDE) // NS
        pltpu.sync_copy(zcnt_hbm.at[pl.ds(sid * cpt, cpt)],
                        acc_cnt.at[pl.ds(sid * cpt, cpt)])
        for i in range(CHUNK // 16):
            ones_v[pl.ds(i * 16, 16)] = jnp.full((16,), 1.0, _f32)
    plsc.subcore_barrier()

    cpp = CHUNKS // phases
    ng = cpp // group

    def fire_gathers(g, buf, sem):
        for b in range(group):
            c = g * group + b
            pltpu.async_copy(tab_sp.at[src_v.at[c]],
                             buf.at[pl.ds(b * CHUNK, CHUNK)], sem)

    def wait_gathers(buf, sem):
        for b in range(group):
            pltpu.make_async_copy(tab_sp.at[src_v.at[0]],
                                  buf.at[pl.ds(b * CHUNK, CHUNK)], sem).wait()

    def fire_scatters(g, buf, sem):
        for b in range(group):
            c = g * group + b
            pltpu.async_copy(buf.at[pl.ds(b * CHUNK, CHUNK)],
                             acc.at[dst_v.at[c]], sem, add=True)
            if do_cnt:
                pltpu.async_copy(ones_v, acc_cnt.at[cidx_v.at[c]], sem,
                                 add=True)

    def wait_scatters(buf, sem):
        for b in range(group):
            pltpu.make_async_copy(buf.at[pl.ds(b * CHUNK, CHUNK)],
                                  acc.at[dst_v.at[0]], sem).wait()
            if do_cnt:
                pltpu.make_async_copy(ones_v, acc_cnt.at[cidx_v.at[0]],
                                      sem).wait()

    for p in range(phases):
        pltpu.sync_copy(src_hbm.at[wid, pl.ds(p * cpp, cpp)], src_v)
        pltpu.sync_copy(dst_hbm.at[wid, pl.ds(p * cpp, cpp)], dst_v)
        if do_cnt:
            pltpu.sync_copy(cidx_hbm.at[wid, pl.ds(p * cpp, cpp)], cidx_v)

        fire_gathers(0, buf0, sem_g0)
        fire_gathers(1, buf1, sem_g1)

        def body(i, carry):
            g = i * 2
            wait_gathers(buf0, sem_g0)
            fire_scatters(g, buf0, sem_s0)
            wait_gathers(buf1, sem_g1)
            wait_scatters(buf0, sem_s0)

            @pl.when(g + 2 < ng)
            def _():
                fire_gathers(g + 2, buf0, sem_g0)

            fire_scatters(g + 1, buf1, sem_s1)
            wait_scatters(buf1, sem_s1)

            @pl.when(g + 3 < ng)
            def _():
                fire_gathers(g + 3, buf1, sem_g1)

            return carry

        lax.fori_loop(0, ng // 2, body, 0)

    plsc.subcore_barrier()
    pltpu.sync_copy(acc.at[pl.ds(sid * rpt, rpt)],
                    out_rows.at[cid, pl.ds(sid * rpt, rpt)])
    if do_cnt:
        pltpu.sync_copy(acc_cnt.at[pl.ds(sid * cpt, cpt)],
                        out_cnt.at[cid, pl.ds(sid * cpt, cpt)])


def _sc_layer1(srcp, dstp, cidxp, g1, zrow, zcnt):
    phases = 4
    cpp = CHUNKS // phases

    def body(src_hbm, dst_hbm, cidx_hbm, tab_hbm, zrow_hbm, zcnt_hbm,
             out_rows, out_cnt, src_v, dst_v, cidx_v, buf0, buf1, ones_v,
             tab_sp, acc, acc_cnt, sem_g0, sem_g1, sem_s0, sem_s1):
        _seg_kernel_body(HID, GROUP1, phases, src_hbm, dst_hbm, cidx_hbm,
                         tab_hbm, zrow_hbm, zcnt_hbm, out_rows, out_cnt,
                         src_v, dst_v, cidx_v, buf0, buf1, ones_v, tab_sp,
                         acc, acc_cnt, sem_g0, sem_g1, sem_s0, sem_s1)

    k = pl.kernel(
        body,
        out_type=(
            jax.ShapeDtypeStruct((NC, NROWS, HID), _f32),
            jax.ShapeDtypeStruct((NC, NROWS * CNT_STRIDE), _f32),
        ),
        mesh=plsc.VectorSubcoreMesh(core_axis_name="c", subcore_axis_name="s"),
        compiler_params=pltpu.CompilerParams(use_tc_tiling_on_sc=False),
        scratch_types=[
            pltpu.VMEM((cpp, CHUNK), jnp.int32),
            pltpu.VMEM((cpp, CHUNK), jnp.int32),
            pltpu.VMEM((cpp, CHUNK), jnp.int32),
            pltpu.VMEM((GROUP1 * CHUNK, HID), _f32),
            pltpu.VMEM((GROUP1 * CHUNK, HID), _f32),
            pltpu.VMEM((CHUNK,), _f32),
            pltpu.VMEM_SHARED((NROWS, HID), _f32),
            pltpu.VMEM_SHARED((NROWS, HID), _f32),
            pltpu.VMEM_SHARED((NROWS * CNT_STRIDE,), _f32),
            pltpu.SemaphoreType.DMA,
            pltpu.SemaphoreType.DMA,
            pltpu.SemaphoreType.DMA,
            pltpu.SemaphoreType.DMA,
        ],
    )

    return k(srcp, dstp, cidxp, g1, zrow, zcnt)


def _sc_layer2(srcp, dstp, g2, zrow):
    def body(src_hbm, dst_hbm, tab_hbm, zrow_hbm, out_rows, src_v, dst_v,
             buf0, buf1, tab_sp, acc, sem_g0, sem_g1, sem_s0, sem_s1):
        _seg_kernel_body(OUT, GROUP2, 1, src_hbm, dst_hbm, None, tab_hbm,
                         zrow_hbm, None, out_rows, None, src_v, dst_v, None,
                         buf0, buf1, None, tab_sp, acc, None,
                         sem_g0, sem_g1, sem_s0, sem_s1)

    k = pl.kernel(
        body,
        out_type=jax.ShapeDtypeStruct((NC, NROWS, OUT), _f32),
        mesh=plsc.VectorSubcoreMesh(core_axis_name="c", subcore_axis_name="s"),
        compiler_params=pltpu.CompilerParams(use_tc_tiling_on_sc=False),
        scratch_types=[
            pltpu.VMEM((CHUNKS, CHUNK), jnp.int32),
            pltpu.VMEM((CHUNKS, CHUNK), jnp.int32),
            pltpu.VMEM((GROUP2 * CHUNK, OUT), _f32),
            pltpu.VMEM((GROUP2 * CHUNK, OUT), _f32),
            pltpu.VMEM_SHARED((NROWS, OUT), _f32),
            pltpu.VMEM_SHARED((NROWS, OUT), _f32),
            pltpu.SemaphoreType.DMA,
            pltpu.SemaphoreType.DMA,
            pltpu.SemaphoreType.DMA,
            pltpu.SemaphoreType.DMA,
        ],
    )
    return k(srcp, dstp, g2, zrow)


def _b_body(p1_ref, cntp_ref, l1_ref, rp1_ref, wi2_ref, wl2_ref, rp2_ref,
            g2_ref, corr2_ref):
    cnt = cntp_ref[0] + cntp_ref[1]
    s = (p1_ref[0] + p1_ref[1] + l1_ref[...]
         - jnp.dot(cnt, rp1_ref[...], preferred_element_type=_f32))
    h1 = jnp.maximum(s, 0.0)
    g2_ref[...] = jnp.dot(h1, wi2_ref[...], preferred_element_type=_f32)
    corr2_ref[...] = (jnp.dot(h1, wl2_ref[...], preferred_element_type=_f32)
                      - jnp.dot(cnt, rp2_ref[...], preferred_element_type=_f32))


def _stage_b(p1, cntp, loop1, relp1, W_in2, W_loop2, relp2):
    return pl.pallas_call(
        _b_body,
        out_shape=(
            jax.ShapeDtypeStruct((NROWS, OUT), _f32),
            jax.ShapeDtypeStruct((NROWS, OUT), _f32),
        ),
    )(p1, cntp, loop1, relp1, W_in2, W_loop2, relp2)


def _c_body(p2_ref, corr2_ref, out_ref):
    out_ref[...] = (p2_ref[0, :N_T, :] + p2_ref[1, :N_T, :]
                    + corr2_ref[:N_T, :])


def _stage_c(p2, corr2):
    return pl.pallas_call(
        _c_body,
        out_shape=jax.ShapeDtypeStruct((N_T, OUT), _f32),
    )(p2, corr2)


def kernel(x_target, x_other, aug_feat_0, aug_feat_1, W_map, b_map, rel1,
           W_in1, W_loop1, W_rel1, W_in2, W_loop2, edge_index, edge_type):
    src = edge_index[0].astype(jnp.int32)
    dst = edge_index[1].astype(jnp.int32)
    et = edge_type.astype(jnp.int32)

    dst2d = dst.reshape(E // 128, 128)
    et2d = et.reshape(E // 128, 128)

    ho, aug, cidx2d = _stage_a1(x_other, W_map, b_map, aug_feat_0,
                                aug_feat_1, dst2d, et2d)
    h = jnp.concatenate(
        [jnp.concatenate([x_target, aug], axis=1), ho], axis=0)
    hp = jnp.concatenate([h, jnp.zeros((NROWS - N, IN_DIM), _f32)], axis=0)

    g1, loop1, relp1, relp2 = _stage_a2(hp, W_in1, W_loop1, rel1, W_rel1,
                                        W_in2)

    pad = EPAD - E
    junk = N + jnp.arange(pad, dtype=jnp.int32) % (NROWS - N)
    srcp = jnp.concatenate([src, jnp.zeros((pad,), jnp.int32)]
                           ).reshape(NW, CHUNKS, CHUNK)
    dstp = jnp.concatenate([dst, junk]).reshape(NW, CHUNKS, CHUNK)
    cidxp = jnp.concatenate([cidx2d.reshape(E), junk * CNT_STRIDE]
                            ).reshape(NW, CHUNKS, CHUNK)

    zrow1 = jnp.zeros((NROWS, HID), _f32)
    zcnt = jnp.zeros((NROWS * CNT_STRIDE,), _f32)
    p1, cntp = _sc_layer1(srcp, dstp, cidxp, g1, zrow1, zcnt)

    g2, corr2 = _stage_b(p1, cntp.reshape(NC, NROWS, CNT_STRIDE), loop1,
                         relp1, W_in2, W_loop2, relp2)

    zrow2 = jnp.zeros((NROWS, OUT), _f32)
    p2 = _sc_layer2(srcp, dstp, g2, zrow2)

    return _stage_c(p2, corr2)


# speedup vs baseline: 1.0794x; 1.0794x over previous
import functools

import jax
import jax.numpy as jnp
from jax import lax
from jax.experimental import pallas as pl
from jax.experimental.pallas import tpu as pltpu
from jax.experimental.pallas import tpu_sc as plsc

N_T = 6000
N_O = 4000
N = N_T + N_O
NROWS = 10240
E = 320000
D_BASE = 128
D_OTHER = 256
EMB = 64
IN_DIM = D_BASE + EMB
HID = 64
OUT = 16
R = 4
CNT_STRIDE = 8

NC, NS = 2, 16
NW = NC * NS
CHUNK = 256
GROUP1 = 1
GROUP2 = 2
CHUNKS = 40
EPW = CHUNKS * CHUNK
EPAD = NW * EPW

_f32 = jnp.float32


EROWS = E // 128
PROWS = EPAD // 128 - EROWS


def _a_body(xt_ref, xo_ref, a0_ref, a1_ref, wmap_ref, bmap_ref, rel1_ref,
            wi1_ref, wl1_ref, wr1_ref, wi2_ref, ei_ref, et_ref,
            g1_ref, l1_ref, rp1_ref, rp2_ref, srcp_ref, dstp_ref, cidxp_ref,
            z1_ref, zc_ref, z2_ref):
    aug = (a0_ref[...] + a1_ref[...]) * 0.5
    xt = xt_ref[...]
    xo = xo_ref[...]
    for wref, out in ((wi1_ref, g1_ref), (wl1_ref, l1_ref)):
        w = wref[...]
        wo = jnp.dot(wmap_ref[...], w, preferred_element_type=_f32)
        bo = jnp.dot(bmap_ref[...][None, :], w, preferred_element_type=_f32)
        out[0:N_T, :] = (jnp.dot(xt, w[0:D_BASE, :],
                                 preferred_element_type=_f32)
                         + jnp.dot(aug, w[D_BASE:, :],
                                   preferred_element_type=_f32))
        out[N_T:N, :] = jnp.dot(xo, wo, preferred_element_type=_f32) + bo
        out[N:, :] = jnp.zeros((NROWS - N, HID), _f32)
    z = jnp.zeros((CNT_STRIDE - R, HID), _f32)
    rp1 = jnp.dot(rel1_ref[...], wi1_ref[...], preferred_element_type=_f32)
    rp1_ref[...] = jnp.concatenate([rp1, z], axis=0)
    rel2 = jnp.dot(rel1_ref[...], wr1_ref[...], preferred_element_type=_f32)
    rp2 = jnp.dot(rel2, wi2_ref[...], preferred_element_type=_f32)
    rp2_ref[...] = jnp.concatenate([rp2, jnp.zeros((CNT_STRIDE - R, OUT), _f32)],
                                   axis=0)
    src2 = ei_ref[0]
    dst2 = ei_ref[1]
    srcp_ref[0:EROWS, :] = src2
    dstp_ref[0:EROWS, :] = dst2
    cidxp_ref[0:EROWS, :] = dst2 * CNT_STRIDE + et_ref[...]
    flat = (lax.broadcasted_iota(jnp.int32, (PROWS, 128), 0) * 128
            + lax.broadcasted_iota(jnp.int32, (PROWS, 128), 1))
    junk = N + lax.rem(flat, NROWS - N)
    srcp_ref[EROWS:, :] = jnp.zeros((PROWS, 128), jnp.int32)
    dstp_ref[EROWS:, :] = junk
    cidxp_ref[EROWS:, :] = junk * CNT_STRIDE
    z1_ref[...] = jnp.zeros((NROWS, HID), _f32)
    zc_ref[...] = jnp.zeros((NROWS * CNT_STRIDE // 128, 128), _f32)
    z2_ref[...] = jnp.zeros((NROWS, OUT), _f32)


def _stage_a(x_target, x_other, a0, a1, W_map, b_map, rel1, W_in1, W_loop1,
             W_rel1, W_in2, ei3, et2):
    return pl.pallas_call(
        _a_body,
        out_shape=(
            jax.ShapeDtypeStruct((NROWS, HID), _f32),
            jax.ShapeDtypeStruct((NROWS, HID), _f32),
            jax.ShapeDtypeStruct((CNT_STRIDE, HID), _f32),
            jax.ShapeDtypeStruct((CNT_STRIDE, OUT), _f32),
            jax.ShapeDtypeStruct((EROWS + PROWS, 128), jnp.int32),
            jax.ShapeDtypeStruct((EROWS + PROWS, 128), jnp.int32),
            jax.ShapeDtypeStruct((EROWS + PROWS, 128), jnp.int32),
            jax.ShapeDtypeStruct((NROWS, HID), _f32),
            jax.ShapeDtypeStruct((NROWS * CNT_STRIDE // 128, 128), _f32),
            jax.ShapeDtypeStruct((NROWS, OUT), _f32),
        ),
    )(x_target, x_other, a0, a1, W_map, b_map, rel1, W_in1, W_loop1,
      W_rel1, W_in2, ei3, et2)


def _seg_kernel_body(width, group, phases, src_hbm, dst_hbm, cidx_hbm,
                     tab_hbm, zrow_hbm, zcnt_hbm, out_rows, out_cnt,
                     src_v, dst_v, cidx_v, buf0, buf1, ones_v, tab_sp,
                     acc, acc_cnt, sem_g0, sem_g1, sem_s0, sem_s1):
    cid = lax.axis_index("c")
    sid = lax.axis_index("s")
    wid = cid * NS + sid
    rpt = NROWS // NS
    pltpu.sync_copy(zrow_hbm.at[pl.ds(sid * rpt, rpt)],
                    acc.at[pl.ds(sid * rpt, rpt)])
    pltpu.sync_copy(tab_hbm.at[pl.ds(sid * rpt, rpt)],
                    tab_sp.at[pl.ds(sid * rpt, rpt)])
    do_cnt = cidx_hbm is not None
    if do_cnt:
        cpt = (NROWS * CNT_STRIDE) // NS
        pltpu.sync_copy(zcnt_hbm.at[pl.ds(sid * cpt, cpt)],
                        acc_cnt.at[pl.ds(sid * cpt, cpt)])
        for i in range(CHUNK // 16):
            ones_v[pl.ds(i * 16, 16)] = jnp.full((16,), 1.0, _f32)
    plsc.subcore_barrier()

    cpp = CHUNKS // phases
    ng = cpp // group

    def fire_gathers(g, buf, sem):
        for b in range(group):
            c = g * group + b
            pltpu.async_copy(tab_sp.at[src_v.at[c]],
                             buf.at[pl.ds(b * CHUNK, CHUNK)], sem)

    def wait_gathers(buf, sem):
        for b in range(group):
            pltpu.make_async_copy(tab_sp.at[src_v.at[0]],
                                  buf.at[pl.ds(b * CHUNK, CHUNK)], sem).wait()

    def fire_scatters(g, buf, sem):
        for b in range(group):
            c = g * group + b
            pltpu.async_copy(buf.at[pl.ds(b * CHUNK, CHUNK)],
                             acc.at[dst_v.at[c]], sem, add=True)
            if do_cnt:
                pltpu.async_copy(ones_v, acc_cnt.at[cidx_v.at[c]], sem,
                                 add=True)

    def wait_scatters(buf, sem):
        for b in range(group):
            pltpu.make_async_copy(buf.at[pl.ds(b * CHUNK, CHUNK)],
                                  acc.at[dst_v.at[0]], sem).wait()
            if do_cnt:
                pltpu.make_async_copy(ones_v, acc_cnt.at[cidx_v.at[0]],
                                      sem).wait()

    for p in range(phases):
        pltpu.sync_copy(src_hbm.at[wid, pl.ds(p * cpp, cpp)], src_v)
        pltpu.sync_copy(dst_hbm.at[wid, pl.ds(p * cpp, cpp)], dst_v)
        if do_cnt:
            pltpu.sync_copy(cidx_hbm.at[wid, pl.ds(p * cpp, cpp)], cidx_v)

        fire_gathers(0, buf0, sem_g0)
        fire_gathers(1, buf1, sem_g1)

        def body(i, carry):
            g = i * 2
            wait_gathers(buf0, sem_g0)
            fire_scatters(g, buf0, sem_s0)
            wait_gathers(buf1, sem_g1)
            wait_scatters(buf0, sem_s0)

            @pl.when(g + 2 < ng)
            def _():
                fire_gathers(g + 2, buf0, sem_g0)

            fire_scatters(g + 1, buf1, sem_s1)
            wait_scatters(buf1, sem_s1)

            @pl.when(g + 3 < ng)
            def _():
                fire_gathers(g + 3, buf1, sem_g1)

            return carry

        lax.fori_loop(0, ng // 2, body, 0)

    plsc.subcore_barrier()
    pltpu.sync_copy(acc.at[pl.ds(sid * rpt, rpt)],
                    out_rows.at[cid, pl.ds(sid * rpt, rpt)])
    if do_cnt:
        pltpu.sync_copy(acc_cnt.at[pl.ds(sid * cpt, cpt)],
                        out_cnt.at[cid, pl.ds(sid * cpt, cpt)])


def _sc_layer1(srcp, dstp, cidxp, g1, zrow, zcnt):
    phases = 4
    cpp = CHUNKS // phases

    def body(src_hbm, dst_hbm, cidx_hbm, tab_hbm, zrow_hbm, zcnt_hbm,
             out_rows, out_cnt, src_v, dst_v, cidx_v, buf0, buf1, ones_v,
             tab_sp, acc, acc_cnt, sem_g0, sem_g1, sem_s0, sem_s1):
        _seg_kernel_body(HID, GROUP1, phases, src_hbm, dst_hbm, cidx_hbm,
                         tab_hbm, zrow_hbm, zcnt_hbm, out_rows, out_cnt,
                         src_v, dst_v, cidx_v, buf0, buf1, ones_v, tab_sp,
                         acc, acc_cnt, sem_g0, sem_g1, sem_s0, sem_s1)

    k = pl.kernel(
        body,
        out_type=(
            jax.ShapeDtypeStruct((NC, NROWS, HID), _f32),
            jax.ShapeDtypeStruct((NC, NROWS * CNT_STRIDE), _f32),
        ),
        mesh=plsc.VectorSubcoreMesh(core_axis_name="c", subcore_axis_name="s"),
        compiler_params=pltpu.CompilerParams(use_tc_tiling_on_sc=False),
        scratch_types=[
            pltpu.VMEM((cpp, CHUNK), jnp.int32),
            pltpu.VMEM((cpp, CHUNK), jnp.int32),
            pltpu.VMEM((cpp, CHUNK), jnp.int32),
            pltpu.VMEM((GROUP1 * CHUNK, HID), _f32),
            pltpu.VMEM((GROUP1 * CHUNK, HID), _f32),
            pltpu.VMEM((CHUNK,), _f32),
            pltpu.VMEM_SHARED((NROWS, HID), _f32),
            pltpu.VMEM_SHARED((NROWS, HID), _f32),
            pltpu.VMEM_SHARED((NROWS * CNT_STRIDE,), _f32),
            pltpu.SemaphoreType.DMA,
            pltpu.SemaphoreType.DMA,
            pltpu.SemaphoreType.DMA,
            pltpu.SemaphoreType.DMA,
        ],
    )

    return k(srcp, dstp, cidxp, g1, zrow, zcnt)


def _sc_layer2(srcp, dstp, g2, zrow):
    def body(src_hbm, dst_hbm, tab_hbm, zrow_hbm, out_rows, src_v, dst_v,
             buf0, buf1, tab_sp, acc, sem_g0, sem_g1, sem_s0, sem_s1):
        _seg_kernel_body(OUT, GROUP2, 1, src_hbm, dst_hbm, None, tab_hbm,
                         zrow_hbm, None, out_rows, None, src_v, dst_v, None,
                         buf0, buf1, None, tab_sp, acc, None,
                         sem_g0, sem_g1, sem_s0, sem_s1)

    k = pl.kernel(
        body,
        out_type=jax.ShapeDtypeStruct((NC, NROWS, OUT), _f32),
        mesh=plsc.VectorSubcoreMesh(core_axis_name="c", subcore_axis_name="s"),
        compiler_params=pltpu.CompilerParams(use_tc_tiling_on_sc=False),
        scratch_types=[
            pltpu.VMEM((CHUNKS, CHUNK), jnp.int32),
            pltpu.VMEM((CHUNKS, CHUNK), jnp.int32),
            pltpu.VMEM((GROUP2 * CHUNK, OUT), _f32),
            pltpu.VMEM((GROUP2 * CHUNK, OUT), _f32),
            pltpu.VMEM_SHARED((NROWS, OUT), _f32),
            pltpu.VMEM_SHARED((NROWS, OUT), _f32),
            pltpu.SemaphoreType.DMA,
            pltpu.SemaphoreType.DMA,
            pltpu.SemaphoreType.DMA,
            pltpu.SemaphoreType.DMA,
        ],
    )
    return k(srcp, dstp, g2, zrow)


def _b0_body(cntp_ref, rp1_ref, rp2_ref, cc1_ref, cc2_ref):
    cnt = cntp_ref[0] + cntp_ref[1]
    bd1 = jnp.tile(rp1_ref[...], (16, 16))
    ii = lax.broadcasted_iota(jnp.int32, (128, 16 * HID), 0)
    jj = lax.broadcasted_iota(jnp.int32, (128, 16 * HID), 1)
    bd1 = jnp.where(ii // CNT_STRIDE == jj // HID, bd1, 0.0)
    cc1_ref[...] = jnp.dot(cnt, bd1, preferred_element_type=_f32)
    bd2 = jnp.tile(rp2_ref[...], (16, 16))
    i2 = lax.broadcasted_iota(jnp.int32, (128, 16 * OUT), 0)
    j2 = lax.broadcasted_iota(jnp.int32, (128, 16 * OUT), 1)
    bd2 = jnp.where(i2 // CNT_STRIDE == j2 // OUT, bd2, 0.0)
    cc2_ref[...] = jnp.dot(cnt, bd2, preferred_element_type=_f32)


def _stage_b0(cntp3, relp1, relp2):
    return pl.pallas_call(
        _b0_body,
        out_shape=(
            jax.ShapeDtypeStruct((NROWS * CNT_STRIDE // 128, 16 * HID), _f32),
            jax.ShapeDtypeStruct((NROWS * CNT_STRIDE // 128, 16 * OUT), _f32),
        ),
    )(cntp3, relp1, relp2)


def _b1_body(p1_ref, l1_ref, cc1_ref, wi2_ref, wl2_ref, cc2_ref,
             g2_ref, corr2_ref):
    h1 = jnp.maximum(p1_ref[0] + p1_ref[1] + l1_ref[...] - cc1_ref[...], 0.0)
    g2_ref[...] = jnp.dot(h1, wi2_ref[...], preferred_element_type=_f32)
    corr2_ref[...] = (jnp.dot(h1, wl2_ref[...], preferred_element_type=_f32)
                      - cc2_ref[...])


def _stage_b1(p1, loop1, cc1, W_in2, W_loop2, cc2):
    return pl.pallas_call(
        _b1_body,
        out_shape=(
            jax.ShapeDtypeStruct((NROWS, OUT), _f32),
            jax.ShapeDtypeStruct((NROWS, OUT), _f32),
        ),
    )(p1, loop1, cc1, W_in2, W_loop2, cc2)


def _c_body(p2_ref, corr2_ref, out_ref):
    out_ref[...] = (p2_ref[0, :N_T, :] + p2_ref[1, :N_T, :]
                    + corr2_ref[:N_T, :])


def _stage_c(p2, corr2):
    return pl.pallas_call(
        _c_body,
        out_shape=jax.ShapeDtypeStruct((N_T, OUT), _f32),
    )(p2, corr2)


def kernel(x_target, x_other, aug_feat_0, aug_feat_1, W_map, b_map, rel1,
           W_in1, W_loop1, W_rel1, W_in2, W_loop2, edge_index, edge_type):
    ei3 = edge_index.astype(jnp.int32).reshape(2, EROWS, 128)
    et2 = edge_type.astype(jnp.int32).reshape(EROWS, 128)

    (g1, loop1, relp1, relp2, srcp2, dstp2, cidxp2, z1, zc, z2) = _stage_a(
        x_target, x_other, aug_feat_0, aug_feat_1, W_map, b_map, rel1,
        W_in1, W_loop1, W_rel1, W_in2, ei3, et2)

    srcp = srcp2.reshape(NW, CHUNKS, CHUNK)
    dstp = dstp2.reshape(NW, CHUNKS, CHUNK)
    cidxp = cidxp2.reshape(NW, CHUNKS, CHUNK)

    p1, cntp = _sc_layer1(srcp, dstp, cidxp, g1, z1,
                          zc.reshape(NROWS * CNT_STRIDE))

    cc1_2d, cc2_2d = _stage_b0(
        cntp.reshape(NC, NROWS * CNT_STRIDE // 128, 128), relp1, relp2)
    g2, corr2 = _stage_b1(p1, loop1, cc1_2d.reshape(NROWS, HID), W_in2,
                          W_loop2, cc2_2d.reshape(NROWS, OUT))

    p2 = _sc_layer2(srcp, dstp, g2, z2)

    return _stage_c(p2, corr2)


# speedup vs baseline: 1.0901x; 1.0099x over previous
import functools

import jax
import jax.numpy as jnp
from jax import lax
from jax.experimental import pallas as pl
from jax.experimental.pallas import tpu as pltpu
from jax.experimental.pallas import tpu_sc as plsc

N_T = 6000
N_O = 4000
N = N_T + N_O
NROWS = 10240
E = 320000
D_BASE = 128
D_OTHER = 256
EMB = 64
IN_DIM = D_BASE + EMB
HID = 64
OUT = 16
R = 4
CNT_STRIDE = 8

NC, NS = 2, 16
NW = NC * NS
CHUNK = 128
GROUP1 = 1
GROUP2 = 4
CHUNKS = 80
EPW = CHUNKS * CHUNK
EPAD = NW * EPW

_f32 = jnp.float32


EROWS = E // 128
PROWS = EPAD // 128 - EROWS


def _a_body(xt_ref, xo_ref, a0_ref, a1_ref, wmap_ref, bmap_ref, rel1_ref,
            wi1_ref, wl1_ref, wr1_ref, wi2_ref, ei_ref, et_ref,
            g1_ref, l1_ref, rp1_ref, rp2_ref, srcp_ref, dstp_ref, cidxp_ref,
            z1_ref, zc_ref, z2_ref):
    aug = (a0_ref[...] + a1_ref[...]) * 0.5
    xt = xt_ref[...]
    xo = xo_ref[...]
    for wref, out in ((wi1_ref, g1_ref), (wl1_ref, l1_ref)):
        w = wref[...]
        wo = jnp.dot(wmap_ref[...], w, preferred_element_type=_f32)
        bo = jnp.dot(bmap_ref[...][None, :], w, preferred_element_type=_f32)
        out[0:N_T, :] = (jnp.dot(xt, w[0:D_BASE, :],
                                 preferred_element_type=_f32)
                         + jnp.dot(aug, w[D_BASE:, :],
                                   preferred_element_type=_f32))
        out[N_T:N, :] = jnp.dot(xo, wo, preferred_element_type=_f32) + bo
        out[N:, :] = jnp.zeros((NROWS - N, HID), _f32)
    z = jnp.zeros((CNT_STRIDE - R, HID), _f32)
    rp1 = jnp.dot(rel1_ref[...], wi1_ref[...], preferred_element_type=_f32)
    rp1_ref[...] = jnp.concatenate([rp1, z], axis=0)
    rel2 = jnp.dot(rel1_ref[...], wr1_ref[...], preferred_element_type=_f32)
    rp2 = jnp.dot(rel2, wi2_ref[...], preferred_element_type=_f32)
    rp2_ref[...] = jnp.concatenate([rp2, jnp.zeros((CNT_STRIDE - R, OUT), _f32)],
                                   axis=0)
    src2 = ei_ref[0]
    dst2 = ei_ref[1]
    srcp_ref[0:EROWS, :] = src2
    dstp_ref[0:EROWS, :] = dst2
    cidxp_ref[0:EROWS, :] = dst2 * CNT_STRIDE + et_ref[...]
    flat = (lax.broadcasted_iota(jnp.int32, (PROWS, 128), 0) * 128
            + lax.broadcasted_iota(jnp.int32, (PROWS, 128), 1))
    junk = N + lax.rem(flat, NROWS - N)
    srcp_ref[EROWS:, :] = jnp.zeros((PROWS, 128), jnp.int32)
    dstp_ref[EROWS:, :] = junk
    cidxp_ref[EROWS:, :] = junk * CNT_STRIDE
    z1_ref[...] = jnp.zeros((NROWS, HID), _f32)
    zc_ref[...] = jnp.zeros((NROWS * CNT_STRIDE // 128, 128), _f32)
    z2_ref[...] = jnp.zeros((NROWS, OUT), _f32)


def _stage_a(x_target, x_other, a0, a1, W_map, b_map, rel1, W_in1, W_loop1,
             W_rel1, W_in2, ei3, et2):
    return pl.pallas_call(
        _a_body,
        out_shape=(
            jax.ShapeDtypeStruct((NROWS, HID), _f32),
            jax.ShapeDtypeStruct((NROWS, HID), _f32),
            jax.ShapeDtypeStruct((CNT_STRIDE, HID), _f32),
            jax.ShapeDtypeStruct((CNT_STRIDE, OUT), _f32),
            jax.ShapeDtypeStruct((EROWS + PROWS, 128), jnp.int32),
            jax.ShapeDtypeStruct((EROWS + PROWS, 128), jnp.int32),
            jax.ShapeDtypeStruct((EROWS + PROWS, 128), jnp.int32),
            jax.ShapeDtypeStruct((NROWS, HID), _f32),
            jax.ShapeDtypeStruct((NROWS * CNT_STRIDE // 128, 128), _f32),
            jax.ShapeDtypeStruct((NROWS, OUT), _f32),
        ),
    )(x_target, x_other, a0, a1, W_map, b_map, rel1, W_in1, W_loop1,
      W_rel1, W_in2, ei3, et2)


def _seg_kernel_body(width, group, phases, src_hbm, dst_hbm, cidx_hbm,
                     tab_hbm, zrow_hbm, zcnt_hbm, out_rows, out_cnt,
                     src_v, dst_v, cidx_v, buf0, buf1, ones_v, tab_sp,
                     acc, acc_cnt, sem_g0, sem_g1, sem_s0, sem_s1):
    cid = lax.axis_index("c")
    sid = lax.axis_index("s")
    wid = cid * NS + sid
    rpt = NROWS // NS
    pltpu.sync_copy(zrow_hbm.at[pl.ds(sid * rpt, rpt)],
                    acc.at[pl.ds(sid * rpt, rpt)])
    pltpu.sync_copy(tab_hbm.at[pl.ds(sid * rpt, rpt)],
                    tab_sp.at[pl.ds(sid * rpt, rpt)])
    do_cnt = cidx_hbm is not None
    if do_cnt:
        cpt = (NROWS * CNT_STRIDE) // NS
        pltpu.sync_copy(zcnt_hbm.at[pl.ds(sid * cpt, cpt)],
                        acc_cnt.at[pl.ds(sid * cpt, cpt)])
        for i in range(CHUNK // 16):
            ones_v[pl.ds(i * 16, 16)] = jnp.full((16,), 1.0, _f32)
    plsc.subcore_barrier()

    cpp = CHUNKS // phases
    ng = cpp // group

    def fire_gathers(g, buf, sem):
        for b in range(group):
            c = g * group + b
            pltpu.async_copy(tab_sp.at[src_v.at[c]],
                             buf.at[pl.ds(b * CHUNK, CHUNK)], sem)

    def wait_gathers(buf, sem):
        for b in range(group):
            pltpu.make_async_copy(tab_sp.at[src_v.at[0]],
                                  buf.at[pl.ds(b * CHUNK, CHUNK)], sem).wait()

    def fire_scatters(g, buf, sem):
        for b in range(group):
            c = g * group + b
            pltpu.async_copy(buf.at[pl.ds(b * CHUNK, CHUNK)],
                             acc.at[dst_v.at[c]], sem, add=True)
            if do_cnt:
                pltpu.async_copy(ones_v, acc_cnt.at[cidx_v.at[c]], sem,
                                 add=True)

    def wait_scatters(buf, sem):
        for b in range(group):
            pltpu.make_async_copy(buf.at[pl.ds(b * CHUNK, CHUNK)],
                                  acc.at[dst_v.at[0]], sem).wait()
            if do_cnt:
                pltpu.make_async_copy(ones_v, acc_cnt.at[cidx_v.at[0]],
                                      sem).wait()

    for p in range(phases):
        pltpu.sync_copy(src_hbm.at[wid, pl.ds(p * cpp, cpp)], src_v)
        pltpu.sync_copy(dst_hbm.at[wid, pl.ds(p * cpp, cpp)], dst_v)
        if do_cnt:
            pltpu.sync_copy(cidx_hbm.at[wid, pl.ds(p * cpp, cpp)], cidx_v)

        fire_gathers(0, buf0, sem_g0)
        fire_gathers(1, buf1, sem_g1)

        def body(i, carry):
            g = i * 2
            wait_gathers(buf0, sem_g0)
            fire_scatters(g, buf0, sem_s0)
            wait_gathers(buf1, sem_g1)
            wait_scatters(buf0, sem_s0)

            @pl.when(g + 2 < ng)
            def _():
                fire_gathers(g + 2, buf0, sem_g0)

            fire_scatters(g + 1, buf1, sem_s1)
            wait_scatters(buf1, sem_s1)

            @pl.when(g + 3 < ng)
            def _():
                fire_gathers(g + 3, buf1, sem_g1)

            return carry

        lax.fori_loop(0, ng // 2, body, 0)

    plsc.subcore_barrier()
    pltpu.sync_copy(acc.at[pl.ds(sid * rpt, rpt)],
                    out_rows.at[cid, pl.ds(sid * rpt, rpt)])
    if do_cnt:
        pltpu.sync_copy(acc_cnt.at[pl.ds(sid * cpt, cpt)],
                        out_cnt.at[cid, pl.ds(sid * cpt, cpt)])


def _sc_layer1(srcp, dstp, cidxp, g1, zrow, zcnt):
    phases = 2
    cpp = CHUNKS // phases

    def body(src_hbm, dst_hbm, cidx_hbm, tab_hbm, zrow_hbm, zcnt_hbm,
             out_rows, out_cnt, src_v, dst_v, cidx_v, buf0, buf1, ones_v,
             tab_sp, acc, acc_cnt, sem_g0, sem_g1, sem_s0, sem_s1):
        _seg_kernel_body(HID, GROUP1, phases, src_hbm, dst_hbm, cidx_hbm,
                         tab_hbm, zrow_hbm, zcnt_hbm, out_rows, out_cnt,
                         src_v, dst_v, cidx_v, buf0, buf1, ones_v, tab_sp,
                         acc, acc_cnt, sem_g0, sem_g1, sem_s0, sem_s1)

    k = pl.kernel(
        body,
        out_type=(
            jax.ShapeDtypeStruct((NC, NROWS, HID), _f32),
            jax.ShapeDtypeStruct((NC, NROWS * CNT_STRIDE), _f32),
        ),
        mesh=plsc.VectorSubcoreMesh(core_axis_name="c", subcore_axis_name="s"),
        compiler_params=pltpu.CompilerParams(use_tc_tiling_on_sc=False),
        scratch_types=[
            pltpu.VMEM((cpp, CHUNK), jnp.int32),
            pltpu.VMEM((cpp, CHUNK), jnp.int32),
            pltpu.VMEM((cpp, CHUNK), jnp.int32),
            pltpu.VMEM((GROUP1 * CHUNK, HID), _f32),
            pltpu.VMEM((GROUP1 * CHUNK, HID), _f32),
            pltpu.VMEM((CHUNK,), _f32),
            pltpu.VMEM_SHARED((NROWS, HID), _f32),
            pltpu.VMEM_SHARED((NROWS, HID), _f32),
            pltpu.VMEM_SHARED((NROWS * CNT_STRIDE,), _f32),
            pltpu.SemaphoreType.DMA,
            pltpu.SemaphoreType.DMA,
            pltpu.SemaphoreType.DMA,
            pltpu.SemaphoreType.DMA,
        ],
    )

    return k(srcp, dstp, cidxp, g1, zrow, zcnt)


def _sc_layer2(srcp, dstp, g2, zrow):
    def body(src_hbm, dst_hbm, tab_hbm, zrow_hbm, out_rows, src_v, dst_v,
             buf0, buf1, tab_sp, acc, sem_g0, sem_g1, sem_s0, sem_s1):
        _seg_kernel_body(OUT, GROUP2, 1, src_hbm, dst_hbm, None, tab_hbm,
                         zrow_hbm, None, out_rows, None, src_v, dst_v, None,
                         buf0, buf1, None, tab_sp, acc, None,
                         sem_g0, sem_g1, sem_s0, sem_s1)

    k = pl.kernel(
        body,
        out_type=jax.ShapeDtypeStruct((NC, NROWS, OUT), _f32),
        mesh=plsc.VectorSubcoreMesh(core_axis_name="c", subcore_axis_name="s"),
        compiler_params=pltpu.CompilerParams(use_tc_tiling_on_sc=False),
        scratch_types=[
            pltpu.VMEM((CHUNKS, CHUNK), jnp.int32),
            pltpu.VMEM((CHUNKS, CHUNK), jnp.int32),
            pltpu.VMEM((GROUP2 * CHUNK, OUT), _f32),
            pltpu.VMEM((GROUP2 * CHUNK, OUT), _f32),
            pltpu.VMEM_SHARED((NROWS, OUT), _f32),
            pltpu.VMEM_SHARED((NROWS, OUT), _f32),
            pltpu.SemaphoreType.DMA,
            pltpu.SemaphoreType.DMA,
            pltpu.SemaphoreType.DMA,
            pltpu.SemaphoreType.DMA,
        ],
    )
    return k(srcp, dstp, g2, zrow)


def _b0_body(cntp_ref, rp1_ref, rp2_ref, cc1_ref, cc2_ref):
    cnt = cntp_ref[0] + cntp_ref[1]
    bd1 = jnp.tile(rp1_ref[...], (16, 16))
    ii = lax.broadcasted_iota(jnp.int32, (128, 16 * HID), 0)
    jj = lax.broadcasted_iota(jnp.int32, (128, 16 * HID), 1)
    bd1 = jnp.where(ii // CNT_STRIDE == jj // HID, bd1, 0.0)
    cc1_ref[...] = jnp.dot(cnt, bd1, preferred_element_type=_f32)
    bd2 = jnp.tile(rp2_ref[...], (16, 16))
    i2 = lax.broadcasted_iota(jnp.int32, (128, 16 * OUT), 0)
    j2 = lax.broadcasted_iota(jnp.int32, (128, 16 * OUT), 1)
    bd2 = jnp.where(i2 // CNT_STRIDE == j2 // OUT, bd2, 0.0)
    cc2_ref[...] = jnp.dot(cnt, bd2, preferred_element_type=_f32)


def _stage_b0(cntp3, relp1, relp2):
    return pl.pallas_call(
        _b0_body,
        out_shape=(
            jax.ShapeDtypeStruct((NROWS * CNT_STRIDE // 128, 16 * HID), _f32),
            jax.ShapeDtypeStruct((NROWS * CNT_STRIDE // 128, 16 * OUT), _f32),
        ),
    )(cntp3, relp1, relp2)


def _b1_body(p1_ref, l1_ref, cc1_ref, wi2_ref, wl2_ref, cc2_ref,
             g2_ref, corr2_ref):
    h1 = jnp.maximum(p1_ref[0] + p1_ref[1] + l1_ref[...] - cc1_ref[...], 0.0)
    g2_ref[...] = jnp.dot(h1, wi2_ref[...], preferred_element_type=_f32)
    corr2_ref[...] = (jnp.dot(h1, wl2_ref[...], preferred_element_type=_f32)
                      - cc2_ref[...])


def _stage_b1(p1, loop1, cc1, W_in2, W_loop2, cc2):
    return pl.pallas_call(
        _b1_body,
        out_shape=(
            jax.ShapeDtypeStruct((NROWS, OUT), _f32),
            jax.ShapeDtypeStruct((NROWS, OUT), _f32),
        ),
    )(p1, loop1, cc1, W_in2, W_loop2, cc2)


def _c_body(p2_ref, corr2_ref, out_ref):
    out_ref[...] = (p2_ref[0, :N_T, :] + p2_ref[1, :N_T, :]
                    + corr2_ref[:N_T, :])


def _stage_c(p2, corr2):
    return pl.pallas_call(
        _c_body,
        out_shape=jax.ShapeDtypeStruct((N_T, OUT), _f32),
    )(p2, corr2)


def kernel(x_target, x_other, aug_feat_0, aug_feat_1, W_map, b_map, rel1,
           W_in1, W_loop1, W_rel1, W_in2, W_loop2, edge_index, edge_type):
    ei3 = edge_index.astype(jnp.int32).reshape(2, EROWS, 128)
    et2 = edge_type.astype(jnp.int32).reshape(EROWS, 128)

    (g1, loop1, relp1, relp2, srcp2, dstp2, cidxp2, z1, zc, z2) = _stage_a(
        x_target, x_other, aug_feat_0, aug_feat_1, W_map, b_map, rel1,
        W_in1, W_loop1, W_rel1, W_in2, ei3, et2)

    srcp = srcp2.reshape(NW, CHUNKS, CHUNK)
    dstp = dstp2.reshape(NW, CHUNKS, CHUNK)
    cidxp = cidxp2.reshape(NW, CHUNKS, CHUNK)

    p1, cntp = _sc_layer1(srcp, dstp, cidxp, g1, z1,
                          zc.reshape(NROWS * CNT_STRIDE))

    cc1_2d, cc2_2d = _stage_b0(
        cntp.reshape(NC, NROWS * CNT_STRIDE // 128, 128), relp1, relp2)
    g2, corr2 = _stage_b1(p1, loop1, cc1_2d.reshape(NROWS, HID), W_in2,
                          W_loop2, cc2_2d.reshape(NROWS, OUT))

    p2 = _sc_layer2(srcp, dstp, g2, z2)

    return _stage_c(p2, corr2)


# speedup vs baseline: 1.2989x; 1.1916x over previous
import functools

import jax
import jax.numpy as jnp
from jax import lax
from jax.experimental import pallas as pl
from jax.experimental.pallas import tpu as pltpu
from jax.experimental.pallas import tpu_sc as plsc

N_T = 6000
N_O = 4000
N = N_T + N_O
NROWS = 10240
E = 320000
D_BASE = 128
D_OTHER = 256
EMB = 64
IN_DIM = D_BASE + EMB
HID = 64
OUT = 16
R = 4
CNT_STRIDE = 8

NC, NS = 2, 16
NW = NC * NS
CHUNK = 128
GROUP1 = 1
GROUP2 = 4
CHUNKS = 80
EPW = CHUNKS * CHUNK
EPAD = NW * EPW

_f32 = jnp.float32


EROWS = E // 128
PROWS = EPAD // 128 - EROWS


def _a_body(xt_ref, xo_ref, a0_ref, a1_ref, wmap_ref, bmap_ref, rel1_ref,
            wi1_ref, wl1_ref, wr1_ref, wi2_ref, ei_ref, et_ref,
            g1_ref, l1_ref, rp1_ref, rp2_ref, srcp_ref, dstp_ref, cidxp_ref,
            z1_ref, zc_ref, z2_ref):
    aug = (a0_ref[...] + a1_ref[...]) * 0.5
    xt = xt_ref[...]
    xo = xo_ref[...]
    for wref, out, odt in ((wi1_ref, g1_ref, jnp.bfloat16),
                           (wl1_ref, l1_ref, _f32)):
        w = wref[...]
        wo = jnp.dot(wmap_ref[...], w, preferred_element_type=_f32)
        bo = jnp.dot(bmap_ref[...][None, :], w, preferred_element_type=_f32)
        out[0:N_T, :] = (jnp.dot(xt, w[0:D_BASE, :],
                                 preferred_element_type=_f32)
                         + jnp.dot(aug, w[D_BASE:, :],
                                   preferred_element_type=_f32)).astype(odt)
        out[N_T:N, :] = (jnp.dot(xo, wo, preferred_element_type=_f32)
                         + bo).astype(odt)
        out[N:, :] = jnp.zeros((NROWS - N, HID), odt)
    z = jnp.zeros((CNT_STRIDE - R, HID), _f32)
    rp1 = jnp.dot(rel1_ref[...], wi1_ref[...], preferred_element_type=_f32)
    rp1_ref[...] = jnp.concatenate([rp1, z], axis=0)
    rel2 = jnp.dot(rel1_ref[...], wr1_ref[...], preferred_element_type=_f32)
    rp2 = jnp.dot(rel2, wi2_ref[...], preferred_element_type=_f32)
    rp2_ref[...] = jnp.concatenate([rp2, jnp.zeros((CNT_STRIDE - R, OUT), _f32)],
                                   axis=0)
    src2 = ei_ref[0]
    dst2 = ei_ref[1]
    srcp_ref[0:EROWS, :] = src2
    dstp_ref[0:EROWS, :] = dst2
    cidxp_ref[0:EROWS, :] = dst2 * CNT_STRIDE + et_ref[...]
    flat = (lax.broadcasted_iota(jnp.int32, (PROWS, 128), 0) * 128
            + lax.broadcasted_iota(jnp.int32, (PROWS, 128), 1))
    junk = N + lax.rem(flat, NROWS - N)
    srcp_ref[EROWS:, :] = jnp.zeros((PROWS, 128), jnp.int32)
    dstp_ref[EROWS:, :] = junk
    cidxp_ref[EROWS:, :] = junk * CNT_STRIDE
    z1_ref[...] = jnp.zeros((NROWS, HID), jnp.bfloat16)
    zc_ref[...] = jnp.zeros((NROWS * CNT_STRIDE // 128, 128), _f32)
    z2_ref[...] = jnp.zeros((NROWS, OUT), _f32)


def _stage_a(x_target, x_other, a0, a1, W_map, b_map, rel1, W_in1, W_loop1,
             W_rel1, W_in2, ei3, et2):
    return pl.pallas_call(
        _a_body,
        out_shape=(
            jax.ShapeDtypeStruct((NROWS, HID), jnp.bfloat16),
            jax.ShapeDtypeStruct((NROWS, HID), _f32),
            jax.ShapeDtypeStruct((CNT_STRIDE, HID), _f32),
            jax.ShapeDtypeStruct((CNT_STRIDE, OUT), _f32),
            jax.ShapeDtypeStruct((EROWS + PROWS, 128), jnp.int32),
            jax.ShapeDtypeStruct((EROWS + PROWS, 128), jnp.int32),
            jax.ShapeDtypeStruct((EROWS + PROWS, 128), jnp.int32),
            jax.ShapeDtypeStruct((NROWS, HID), jnp.bfloat16),
            jax.ShapeDtypeStruct((NROWS * CNT_STRIDE // 128, 128), _f32),
            jax.ShapeDtypeStruct((NROWS, OUT), _f32),
        ),
    )(x_target, x_other, a0, a1, W_map, b_map, rel1, W_in1, W_loop1,
      W_rel1, W_in2, ei3, et2)


def _seg_kernel_body(width, group, phases, src_hbm, dst_hbm, cidx_hbm,
                     tab_hbm, zrow_hbm, zcnt_hbm, out_rows, out_cnt,
                     src_v, dst_v, cidx_v, buf0, buf1, ones_v, tab_sp,
                     acc, acc_cnt, sem_g0, sem_g1, sem_s0, sem_s1):
    cid = lax.axis_index("c")
    sid = lax.axis_index("s")
    wid = cid * NS + sid
    rpt = NROWS // NS
    pltpu.sync_copy(zrow_hbm.at[pl.ds(sid * rpt, rpt)],
                    acc.at[pl.ds(sid * rpt, rpt)])
    pltpu.sync_copy(tab_hbm.at[pl.ds(sid * rpt, rpt)],
                    tab_sp.at[pl.ds(sid * rpt, rpt)])
    do_cnt = cidx_hbm is not None
    if do_cnt:
        cpt = (NROWS * CNT_STRIDE) // NS
        pltpu.sync_copy(zcnt_hbm.at[pl.ds(sid * cpt, cpt)],
                        acc_cnt.at[pl.ds(sid * cpt, cpt)])
        for i in range(CHUNK // 16):
            ones_v[pl.ds(i * 16, 16)] = jnp.full((16,), 1.0, _f32)
    plsc.subcore_barrier()

    cpp = CHUNKS // phases
    ng = cpp // group

    def fire_gathers(g, buf, sem):
        for b in range(group):
            c = g * group + b
            pltpu.async_copy(tab_sp.at[src_v.at[c]],
                             buf.at[pl.ds(b * CHUNK, CHUNK)], sem)

    def wait_gathers(buf, sem):
        for b in range(group):
            pltpu.make_async_copy(tab_sp.at[src_v.at[0]],
                                  buf.at[pl.ds(b * CHUNK, CHUNK)], sem).wait()

    def fire_scatters(g, buf, sem):
        for b in range(group):
            c = g * group + b
            pltpu.async_copy(buf.at[pl.ds(b * CHUNK, CHUNK)],
                             acc.at[dst_v.at[c]], sem, add=True)
            if do_cnt:
                pltpu.async_copy(ones_v, acc_cnt.at[cidx_v.at[c]], sem,
                                 add=True)

    def wait_scatters(buf, sem):
        for b in range(group):
            pltpu.make_async_copy(buf.at[pl.ds(b * CHUNK, CHUNK)],
                                  acc.at[dst_v.at[0]], sem).wait()
            if do_cnt:
                pltpu.make_async_copy(ones_v, acc_cnt.at[cidx_v.at[0]],
                                      sem).wait()

    for p in range(phases):
        pltpu.sync_copy(src_hbm.at[wid, pl.ds(p * cpp, cpp)], src_v)
        pltpu.sync_copy(dst_hbm.at[wid, pl.ds(p * cpp, cpp)], dst_v)
        if do_cnt:
            pltpu.sync_copy(cidx_hbm.at[wid, pl.ds(p * cpp, cpp)], cidx_v)

        fire_gathers(0, buf0, sem_g0)
        fire_gathers(1, buf1, sem_g1)

        def body(i, carry):
            g = i * 2
            wait_gathers(buf0, sem_g0)
            fire_scatters(g, buf0, sem_s0)
            wait_gathers(buf1, sem_g1)
            wait_scatters(buf0, sem_s0)

            @pl.when(g + 2 < ng)
            def _():
                fire_gathers(g + 2, buf0, sem_g0)

            fire_scatters(g + 1, buf1, sem_s1)
            wait_scatters(buf1, sem_s1)

            @pl.when(g + 3 < ng)
            def _():
                fire_gathers(g + 3, buf1, sem_g1)

            return carry

        lax.fori_loop(0, ng // 2, body, 0)

    plsc.subcore_barrier()
    pltpu.sync_copy(acc.at[pl.ds(sid * rpt, rpt)],
                    out_rows.at[cid, pl.ds(sid * rpt, rpt)])
    if do_cnt:
        pltpu.sync_copy(acc_cnt.at[pl.ds(sid * cpt, cpt)],
                        out_cnt.at[cid, pl.ds(sid * cpt, cpt)])


def _sc_layer1(srcp, dstp, cidxp, g1, zrow, zcnt):
    phases = 2
    cpp = CHUNKS // phases

    def body(src_hbm, dst_hbm, cidx_hbm, tab_hbm, zrow_hbm, zcnt_hbm,
             out_rows, out_cnt, src_v, dst_v, cidx_v, buf0, buf1, ones_v,
             tab_sp, acc, acc_cnt, sem_g0, sem_g1, sem_s0, sem_s1):
        _seg_kernel_body(HID, GROUP1, phases, src_hbm, dst_hbm, cidx_hbm,
                         tab_hbm, zrow_hbm, zcnt_hbm, out_rows, out_cnt,
                         src_v, dst_v, cidx_v, buf0, buf1, ones_v, tab_sp,
                         acc, acc_cnt, sem_g0, sem_g1, sem_s0, sem_s1)

    k = pl.kernel(
        body,
        out_type=(
            jax.ShapeDtypeStruct((NC, NROWS, HID), jnp.bfloat16),
            jax.ShapeDtypeStruct((NC, NROWS * CNT_STRIDE), _f32),
        ),
        mesh=plsc.VectorSubcoreMesh(core_axis_name="c", subcore_axis_name="s"),
        compiler_params=pltpu.CompilerParams(use_tc_tiling_on_sc=False),
        scratch_types=[
            pltpu.VMEM((cpp, CHUNK), jnp.int32),
            pltpu.VMEM((cpp, CHUNK), jnp.int32),
            pltpu.VMEM((cpp, CHUNK), jnp.int32),
            pltpu.VMEM((GROUP1 * CHUNK, HID), jnp.bfloat16),
            pltpu.VMEM((GROUP1 * CHUNK, HID), jnp.bfloat16),
            pltpu.VMEM((CHUNK,), _f32),
            pltpu.VMEM_SHARED((NROWS, HID), jnp.bfloat16),
            pltpu.VMEM_SHARED((NROWS, HID), jnp.bfloat16),
            pltpu.VMEM_SHARED((NROWS * CNT_STRIDE,), _f32),
            pltpu.SemaphoreType.DMA,
            pltpu.SemaphoreType.DMA,
            pltpu.SemaphoreType.DMA,
            pltpu.SemaphoreType.DMA,
        ],
    )

    return k(srcp, dstp, cidxp, g1, zrow, zcnt)


def _sc_layer2(srcp, dstp, g2, zrow):
    def body(src_hbm, dst_hbm, tab_hbm, zrow_hbm, out_rows, src_v, dst_v,
             buf0, buf1, tab_sp, acc, sem_g0, sem_g1, sem_s0, sem_s1):
        _seg_kernel_body(OUT, GROUP2, 1, src_hbm, dst_hbm, None, tab_hbm,
                         zrow_hbm, None, out_rows, None, src_v, dst_v, None,
                         buf0, buf1, None, tab_sp, acc, None,
                         sem_g0, sem_g1, sem_s0, sem_s1)

    k = pl.kernel(
        body,
        out_type=jax.ShapeDtypeStruct((NC, NROWS, OUT), _f32),
        mesh=plsc.VectorSubcoreMesh(core_axis_name="c", subcore_axis_name="s"),
        compiler_params=pltpu.CompilerParams(use_tc_tiling_on_sc=False),
        scratch_types=[
            pltpu.VMEM((CHUNKS, CHUNK), jnp.int32),
            pltpu.VMEM((CHUNKS, CHUNK), jnp.int32),
            pltpu.VMEM((GROUP2 * CHUNK, OUT), _f32),
            pltpu.VMEM((GROUP2 * CHUNK, OUT), _f32),
            pltpu.VMEM_SHARED((NROWS, OUT), _f32),
            pltpu.VMEM_SHARED((NROWS, OUT), _f32),
            pltpu.SemaphoreType.DMA,
            pltpu.SemaphoreType.DMA,
            pltpu.SemaphoreType.DMA,
            pltpu.SemaphoreType.DMA,
        ],
    )
    return k(srcp, dstp, g2, zrow)


def _b0_body(cntp_ref, rp1_ref, rp2_ref, cc1_ref, cc2_ref):
    cnt = cntp_ref[0] + cntp_ref[1]
    bd1 = jnp.tile(rp1_ref[...], (16, 16))
    ii = lax.broadcasted_iota(jnp.int32, (128, 16 * HID), 0)
    jj = lax.broadcasted_iota(jnp.int32, (128, 16 * HID), 1)
    bd1 = jnp.where(ii // CNT_STRIDE == jj // HID, bd1, 0.0)
    cc1_ref[...] = jnp.dot(cnt, bd1, preferred_element_type=_f32)
    bd2 = jnp.tile(rp2_ref[...], (16, 16))
    i2 = lax.broadcasted_iota(jnp.int32, (128, 16 * OUT), 0)
    j2 = lax.broadcasted_iota(jnp.int32, (128, 16 * OUT), 1)
    bd2 = jnp.where(i2 // CNT_STRIDE == j2 // OUT, bd2, 0.0)
    cc2_ref[...] = jnp.dot(cnt, bd2, preferred_element_type=_f32)


def _stage_b0(cntp3, relp1, relp2):
    return pl.pallas_call(
        _b0_body,
        out_shape=(
            jax.ShapeDtypeStruct((NROWS * CNT_STRIDE // 128, 16 * HID), _f32),
            jax.ShapeDtypeStruct((NROWS * CNT_STRIDE // 128, 16 * OUT), _f32),
        ),
    )(cntp3, relp1, relp2)


def _b1_body(p1_ref, l1_ref, cc1_ref, wi2_ref, wl2_ref, cc2_ref,
             g2_ref, corr2_ref):
    h1 = jnp.maximum(p1_ref[0].astype(_f32) + p1_ref[1].astype(_f32)
                     + l1_ref[...] - cc1_ref[...], 0.0)
    g2_ref[...] = jnp.dot(h1, wi2_ref[...], preferred_element_type=_f32)
    corr2_ref[...] = (jnp.dot(h1, wl2_ref[...], preferred_element_type=_f32)
                      - cc2_ref[...])


def _stage_b1(p1, loop1, cc1, W_in2, W_loop2, cc2):
    return pl.pallas_call(
        _b1_body,
        out_shape=(
            jax.ShapeDtypeStruct((NROWS, OUT), _f32),
            jax.ShapeDtypeStruct((NROWS, OUT), _f32),
        ),
    )(p1, loop1, cc1, W_in2, W_loop2, cc2)


def _c_body(p2_ref, corr2_ref, out_ref):
    out_ref[...] = (p2_ref[0, :N_T, :] + p2_ref[1, :N_T, :]
                    + corr2_ref[:N_T, :])


def _stage_c(p2, corr2):
    return pl.pallas_call(
        _c_body,
        out_shape=jax.ShapeDtypeStruct((N_T, OUT), _f32),
    )(p2, corr2)


def kernel(x_target, x_other, aug_feat_0, aug_feat_1, W_map, b_map, rel1,
           W_in1, W_loop1, W_rel1, W_in2, W_loop2, edge_index, edge_type):
    ei3 = edge_index.astype(jnp.int32).reshape(2, EROWS, 128)
    et2 = edge_type.astype(jnp.int32).reshape(EROWS, 128)

    (g1, loop1, relp1, relp2, srcp2, dstp2, cidxp2, z1, zc, z2) = _stage_a(
        x_target, x_other, aug_feat_0, aug_feat_1, W_map, b_map, rel1,
        W_in1, W_loop1, W_rel1, W_in2, ei3, et2)

    srcp = srcp2.reshape(NW, CHUNKS, CHUNK)
    dstp = dstp2.reshape(NW, CHUNKS, CHUNK)
    cidxp = cidxp2.reshape(NW, CHUNKS, CHUNK)

    p1, cntp = _sc_layer1(srcp, dstp, cidxp, g1, z1,
                          zc.reshape(NROWS * CNT_STRIDE))

    cc1_2d, cc2_2d = _stage_b0(
        cntp.reshape(NC, NROWS * CNT_STRIDE // 128, 128), relp1, relp2)
    g2, corr2 = _stage_b1(p1, loop1, cc1_2d.reshape(NROWS, HID), W_in2,
                          W_loop2, cc2_2d.reshape(NROWS, OUT))

    p2 = _sc_layer2(srcp, dstp, g2, z2)

    return _stage_c(p2, corr2)


# speedup vs baseline: 1.3678x; 1.0530x over previous
import functools

import jax
import jax.numpy as jnp
from jax import lax
from jax.experimental import pallas as pl
from jax.experimental.pallas import tpu as pltpu
from jax.experimental.pallas import tpu_sc as plsc

N_T = 6000
N_O = 4000
N = N_T + N_O
NROWS = 10240
E = 320000
D_BASE = 128
D_OTHER = 256
EMB = 64
IN_DIM = D_BASE + EMB
HID = 64
OUT = 16
R = 4
CNT_STRIDE = 8

NC, NS = 2, 16
NW = NC * NS
CHUNK = 128
GROUP1 = 4
GROUP2 = 4
CHUNKS = 80
EPW = CHUNKS * CHUNK
EPAD = NW * EPW

_f32 = jnp.float32


EROWS = E // 128
PROWS = EPAD // 128 - EROWS


def _a_body(xt_ref, xo_ref, a0_ref, a1_ref, wmap_ref, bmap_ref, rel1_ref,
            wi1_ref, wl1_ref, wr1_ref, wi2_ref, ei_ref, et_ref,
            g1_ref, l1_ref, rp1_ref, rp2_ref, srcp_ref, dstp_ref, cidxp_ref,
            z1_ref, zc_ref, z2_ref):
    aug = (a0_ref[...] + a1_ref[...]) * 0.5
    xt = xt_ref[...]
    xo = xo_ref[...]
    for wref, out, odt in ((wi1_ref, g1_ref, jnp.bfloat16),
                           (wl1_ref, l1_ref, _f32)):
        w = wref[...]
        wo = jnp.dot(wmap_ref[...], w, preferred_element_type=_f32)
        bo = jnp.dot(bmap_ref[...][None, :], w, preferred_element_type=_f32)
        out[0:N_T, :] = (jnp.dot(xt, w[0:D_BASE, :],
                                 preferred_element_type=_f32)
                         + jnp.dot(aug, w[D_BASE:, :],
                                   preferred_element_type=_f32)).astype(odt)
        out[N_T:N, :] = (jnp.dot(xo, wo, preferred_element_type=_f32)
                         + bo).astype(odt)
        out[N:, :] = jnp.zeros((NROWS - N, HID), odt)
    z = jnp.zeros((CNT_STRIDE - R, HID), _f32)
    rp1 = jnp.dot(rel1_ref[...], wi1_ref[...], preferred_element_type=_f32)
    rp1_ref[...] = jnp.concatenate([rp1, z], axis=0)
    rel2 = jnp.dot(rel1_ref[...], wr1_ref[...], preferred_element_type=_f32)
    rp2 = jnp.dot(rel2, wi2_ref[...], preferred_element_type=_f32)
    rp2_ref[...] = jnp.concatenate([rp2, jnp.zeros((CNT_STRIDE - R, OUT), _f32)],
                                   axis=0)
    src2 = ei_ref[0]
    dst2 = ei_ref[1]
    srcp_ref[0:EROWS, :] = src2
    dstp_ref[0:EROWS, :] = dst2
    cidxp_ref[0:EROWS, :] = dst2 * CNT_STRIDE + et_ref[...]
    flat = (lax.broadcasted_iota(jnp.int32, (PROWS, 128), 0) * 128
            + lax.broadcasted_iota(jnp.int32, (PROWS, 128), 1))
    junk = N + lax.rem(flat, NROWS - N)
    srcp_ref[EROWS:, :] = jnp.zeros((PROWS, 128), jnp.int32)
    dstp_ref[EROWS:, :] = junk
    cidxp_ref[EROWS:, :] = junk * CNT_STRIDE
    z1_ref[...] = jnp.zeros((NROWS, HID), jnp.bfloat16)
    zc_ref[...] = jnp.zeros((NROWS * CNT_STRIDE // 128, 128), _f32)
    z2_ref[...] = jnp.zeros((NROWS, OUT), jnp.bfloat16)


def _stage_a(x_target, x_other, a0, a1, W_map, b_map, rel1, W_in1, W_loop1,
             W_rel1, W_in2, ei3, et2):
    return pl.pallas_call(
        _a_body,
        out_shape=(
            jax.ShapeDtypeStruct((NROWS, HID), jnp.bfloat16),
            jax.ShapeDtypeStruct((NROWS, HID), _f32),
            jax.ShapeDtypeStruct((CNT_STRIDE, HID), _f32),
            jax.ShapeDtypeStruct((CNT_STRIDE, OUT), _f32),
            jax.ShapeDtypeStruct((EROWS + PROWS, 128), jnp.int32),
            jax.ShapeDtypeStruct((EROWS + PROWS, 128), jnp.int32),
            jax.ShapeDtypeStruct((EROWS + PROWS, 128), jnp.int32),
            jax.ShapeDtypeStruct((NROWS, HID), jnp.bfloat16),
            jax.ShapeDtypeStruct((NROWS * CNT_STRIDE // 128, 128), _f32),
            jax.ShapeDtypeStruct((NROWS, OUT), jnp.bfloat16),
        ),
    )(x_target, x_other, a0, a1, W_map, b_map, rel1, W_in1, W_loop1,
      W_rel1, W_in2, ei3, et2)


def _seg_kernel_body(width, group, phases, src_hbm, dst_hbm, cidx_hbm,
                     tab_hbm, zrow_hbm, zcnt_hbm, out_rows, out_cnt,
                     src_v, dst_v, cidx_v, buf0, buf1, ones_v, tab_sp,
                     acc, acc_cnt, sem_g0, sem_g1, sem_s0, sem_s1):
    cid = lax.axis_index("c")
    sid = lax.axis_index("s")
    wid = cid * NS + sid
    rpt = NROWS // NS
    pltpu.sync_copy(zrow_hbm.at[pl.ds(sid * rpt, rpt)],
                    acc.at[pl.ds(sid * rpt, rpt)])
    pltpu.sync_copy(tab_hbm.at[pl.ds(sid * rpt, rpt)],
                    tab_sp.at[pl.ds(sid * rpt, rpt)])
    do_cnt = cidx_hbm is not None
    if do_cnt:
        cpt = (NROWS * CNT_STRIDE) // NS
        pltpu.sync_copy(zcnt_hbm.at[pl.ds(sid * cpt, cpt)],
                        acc_cnt.at[pl.ds(sid * cpt, cpt)])
        for i in range(CHUNK // 16):
            ones_v[pl.ds(i * 16, 16)] = jnp.full((16,), 1.0, _f32)
    plsc.subcore_barrier()

    cpp = CHUNKS // phases
    ng = cpp // group

    def fire_gathers(g, buf, sem):
        for b in range(group):
            c = g * group + b
            pltpu.async_copy(tab_sp.at[src_v.at[c]],
                             buf.at[pl.ds(b * CHUNK, CHUNK)], sem)

    def wait_gathers(buf, sem):
        for b in range(group):
            pltpu.make_async_copy(tab_sp.at[src_v.at[0]],
                                  buf.at[pl.ds(b * CHUNK, CHUNK)], sem).wait()

    def fire_scatters(g, buf, sem):
        for b in range(group):
            c = g * group + b
            pltpu.async_copy(buf.at[pl.ds(b * CHUNK, CHUNK)],
                             acc.at[dst_v.at[c]], sem, add=True)
            if do_cnt:
                pltpu.async_copy(ones_v, acc_cnt.at[cidx_v.at[c]], sem,
                                 add=True)

    def wait_scatters(buf, sem):
        for b in range(group):
            pltpu.make_async_copy(buf.at[pl.ds(b * CHUNK, CHUNK)],
                                  acc.at[dst_v.at[0]], sem).wait()
            if do_cnt:
                pltpu.make_async_copy(ones_v, acc_cnt.at[cidx_v.at[0]],
                                      sem).wait()

    for p in range(phases):
        pltpu.sync_copy(src_hbm.at[wid, pl.ds(p * cpp, cpp)], src_v)
        pltpu.sync_copy(dst_hbm.at[wid, pl.ds(p * cpp, cpp)], dst_v)
        if do_cnt:
            pltpu.sync_copy(cidx_hbm.at[wid, pl.ds(p * cpp, cpp)], cidx_v)

        fire_gathers(0, buf0, sem_g0)
        fire_gathers(1, buf1, sem_g1)

        def body(i, carry):
            g = i * 2
            wait_gathers(buf0, sem_g0)
            fire_scatters(g, buf0, sem_s0)
            wait_gathers(buf1, sem_g1)
            wait_scatters(buf0, sem_s0)

            @pl.when(g + 2 < ng)
            def _():
                fire_gathers(g + 2, buf0, sem_g0)

            fire_scatters(g + 1, buf1, sem_s1)
            wait_scatters(buf1, sem_s1)

            @pl.when(g + 3 < ng)
            def _():
                fire_gathers(g + 3, buf1, sem_g1)

            return carry

        lax.fori_loop(0, ng // 2, body, 0)

    plsc.subcore_barrier()
    pltpu.sync_copy(acc.at[pl.ds(sid * rpt, rpt)],
                    out_rows.at[cid, pl.ds(sid * rpt, rpt)])
    if do_cnt:
        pltpu.sync_copy(acc_cnt.at[pl.ds(sid * cpt, cpt)],
                        out_cnt.at[cid, pl.ds(sid * cpt, cpt)])


def _sc_layer1(srcp, dstp, cidxp, g1, zrow, zcnt):
    phases = 1
    cpp = CHUNKS // phases

    def body(src_hbm, dst_hbm, cidx_hbm, tab_hbm, zrow_hbm, zcnt_hbm,
             out_rows, out_cnt, src_v, dst_v, cidx_v, buf0, buf1, ones_v,
             tab_sp, acc, acc_cnt, sem_g0, sem_g1, sem_s0, sem_s1):
        _seg_kernel_body(HID, GROUP1, phases, src_hbm, dst_hbm, cidx_hbm,
                         tab_hbm, zrow_hbm, zcnt_hbm, out_rows, out_cnt,
                         src_v, dst_v, cidx_v, buf0, buf1, ones_v, tab_sp,
                         acc, acc_cnt, sem_g0, sem_g1, sem_s0, sem_s1)

    k = pl.kernel(
        body,
        out_type=(
            jax.ShapeDtypeStruct((NC, NROWS, HID), jnp.bfloat16),
            jax.ShapeDtypeStruct((NC, NROWS * CNT_STRIDE), _f32),
        ),
        mesh=plsc.VectorSubcoreMesh(core_axis_name="c", subcore_axis_name="s"),
        compiler_params=pltpu.CompilerParams(use_tc_tiling_on_sc=False),
        scratch_types=[
            pltpu.VMEM((CHUNKS, CHUNK), jnp.int32),
            pltpu.VMEM((CHUNKS, CHUNK), jnp.int32),
            pltpu.VMEM((CHUNKS, CHUNK), jnp.int32),
            pltpu.VMEM((GROUP1 * CHUNK, HID), jnp.bfloat16),
            pltpu.VMEM((GROUP1 * CHUNK, HID), jnp.bfloat16),
            pltpu.VMEM((CHUNK,), _f32),
            pltpu.VMEM_SHARED((NROWS, HID), jnp.bfloat16),
            pltpu.VMEM_SHARED((NROWS, HID), jnp.bfloat16),
            pltpu.VMEM_SHARED((NROWS * CNT_STRIDE,), _f32),
            pltpu.SemaphoreType.DMA,
            pltpu.SemaphoreType.DMA,
            pltpu.SemaphoreType.DMA,
            pltpu.SemaphoreType.DMA,
        ],
    )

    return k(srcp, dstp, cidxp, g1, zrow, zcnt)


def _sc_layer2(srcp, dstp, g2, zrow):
    def body(src_hbm, dst_hbm, tab_hbm, zrow_hbm, out_rows, src_v, dst_v,
             buf0, buf1, tab_sp, acc, sem_g0, sem_g1, sem_s0, sem_s1):
        _seg_kernel_body(OUT, GROUP2, 1, src_hbm, dst_hbm, None, tab_hbm,
                         zrow_hbm, None, out_rows, None, src_v, dst_v, None,
                         buf0, buf1, None, tab_sp, acc, None,
                         sem_g0, sem_g1, sem_s0, sem_s1)

    k = pl.kernel(
        body,
        out_type=jax.ShapeDtypeStruct((NC, NROWS, OUT), jnp.bfloat16),
        mesh=plsc.VectorSubcoreMesh(core_axis_name="c", subcore_axis_name="s"),
        compiler_params=pltpu.CompilerParams(use_tc_tiling_on_sc=False),
        scratch_types=[
            pltpu.VMEM((CHUNKS, CHUNK), jnp.int32),
            pltpu.VMEM((CHUNKS, CHUNK), jnp.int32),
            pltpu.VMEM((GROUP2 * CHUNK, OUT), jnp.bfloat16),
            pltpu.VMEM((GROUP2 * CHUNK, OUT), jnp.bfloat16),
            pltpu.VMEM_SHARED((NROWS, OUT), jnp.bfloat16),
            pltpu.VMEM_SHARED((NROWS, OUT), jnp.bfloat16),
            pltpu.SemaphoreType.DMA,
            pltpu.SemaphoreType.DMA,
            pltpu.SemaphoreType.DMA,
            pltpu.SemaphoreType.DMA,
        ],
    )
    return k(srcp, dstp, g2, zrow)


def _b0_body(cntp_ref, rp1_ref, rp2_ref, cc1_ref, cc2_ref):
    cnt = cntp_ref[0] + cntp_ref[1]
    bd1 = jnp.tile(rp1_ref[...], (16, 16))
    ii = lax.broadcasted_iota(jnp.int32, (128, 16 * HID), 0)
    jj = lax.broadcasted_iota(jnp.int32, (128, 16 * HID), 1)
    bd1 = jnp.where(ii // CNT_STRIDE == jj // HID, bd1, 0.0)
    cc1_ref[...] = jnp.dot(cnt, bd1, preferred_element_type=_f32)
    bd2 = jnp.tile(rp2_ref[...], (16, 16))
    i2 = lax.broadcasted_iota(jnp.int32, (128, 16 * OUT), 0)
    j2 = lax.broadcasted_iota(jnp.int32, (128, 16 * OUT), 1)
    bd2 = jnp.where(i2 // CNT_STRIDE == j2 // OUT, bd2, 0.0)
    cc2_ref[...] = jnp.dot(cnt, bd2, preferred_element_type=_f32)


def _stage_b0(cntp3, relp1, relp2):
    return pl.pallas_call(
        _b0_body,
        out_shape=(
            jax.ShapeDtypeStruct((NROWS * CNT_STRIDE // 128, 16 * HID), _f32),
            jax.ShapeDtypeStruct((NROWS * CNT_STRIDE // 128, 16 * OUT), _f32),
        ),
    )(cntp3, relp1, relp2)


def _b1_body(p1_ref, l1_ref, cc1_ref, wi2_ref, wl2_ref, cc2_ref,
             g2_ref, corr2_ref):
    h1 = jnp.maximum(p1_ref[0].astype(_f32) + p1_ref[1].astype(_f32)
                     + l1_ref[...] - cc1_ref[...], 0.0)
    g2_ref[...] = jnp.dot(h1, wi2_ref[...],
                          preferred_element_type=_f32).astype(jnp.bfloat16)
    corr2_ref[...] = (jnp.dot(h1, wl2_ref[...], preferred_element_type=_f32)
                      - cc2_ref[...])


def _stage_b1(p1, loop1, cc1, W_in2, W_loop2, cc2):
    return pl.pallas_call(
        _b1_body,
        out_shape=(
            jax.ShapeDtypeStruct((NROWS, OUT), jnp.bfloat16),
            jax.ShapeDtypeStruct((NROWS, OUT), _f32),
        ),
    )(p1, loop1, cc1, W_in2, W_loop2, cc2)


def _c_body(p2_ref, corr2_ref, out_ref):
    out_ref[...] = (p2_ref[0, :N_T, :].astype(_f32)
                    + p2_ref[1, :N_T, :].astype(_f32)
                    + corr2_ref[:N_T, :])


def _stage_c(p2, corr2):
    return pl.pallas_call(
        _c_body,
        out_shape=jax.ShapeDtypeStruct((N_T, OUT), _f32),
    )(p2, corr2)


def kernel(x_target, x_other, aug_feat_0, aug_feat_1, W_map, b_map, rel1,
           W_in1, W_loop1, W_rel1, W_in2, W_loop2, edge_index, edge_type):
    ei3 = edge_index.astype(jnp.int32).reshape(2, EROWS, 128)
    et2 = edge_type.astype(jnp.int32).reshape(EROWS, 128)

    (g1, loop1, relp1, relp2, srcp2, dstp2, cidxp2, z1, zc, z2) = _stage_a(
        x_target, x_other, aug_feat_0, aug_feat_1, W_map, b_map, rel1,
        W_in1, W_loop1, W_rel1, W_in2, ei3, et2)

    srcp = srcp2.reshape(NW, CHUNKS, CHUNK)
    dstp = dstp2.reshape(NW, CHUNKS, CHUNK)
    cidxp = cidxp2.reshape(NW, CHUNKS, CHUNK)

    p1, cntp = _sc_layer1(srcp, dstp, cidxp, g1, z1,
                          zc.reshape(NROWS * CNT_STRIDE))

    cc1_2d, cc2_2d = _stage_b0(
        cntp.reshape(NC, NROWS * CNT_STRIDE // 128, 128), relp1, relp2)
    g2, corr2 = _stage_b1(p1, loop1, cc1_2d.reshape(NROWS, HID), W_in2,
                          W_loop2, cc2_2d.reshape(NROWS, OUT))

    p2 = _sc_layer2(srcp, dstp, g2, z2)

    return _stage_c(p2, corr2)


# speedup vs baseline: 1.3848x; 1.0125x over previous
import functools

import jax
import jax.numpy as jnp
from jax import lax
from jax.experimental import pallas as pl
from jax.experimental.pallas import tpu as pltpu
from jax.experimental.pallas import tpu_sc as plsc

N_T = 6000
N_O = 4000
N = N_T + N_O
NROWS = 10240
E = 320000
D_BASE = 128
D_OTHER = 256
EMB = 64
IN_DIM = D_BASE + EMB
HID = 64
OUT = 16
R = 4
CNT_STRIDE = 8

NC, NS = 2, 16
NW = NC * NS
CHUNK = 128
GROUP1 = 4
GROUP2 = 4
CHUNKS = 80
EPW = CHUNKS * CHUNK
EPAD = NW * EPW

_f32 = jnp.float32


EROWS = E // 128
PROWS = EPAD // 128 - EROWS


def _a_body(xt_ref, xo_ref, a0_ref, a1_ref, wmap_ref, bmap_ref, rel1_ref,
            wi1_ref, wl1_ref, wr1_ref, wi2_ref, ei_ref, et_ref,
            g1_ref, l1_ref, rp1_ref, rp2_ref, srcp_ref, dstp_ref, cidxp_ref,
            z1_ref, zc_ref, z2_ref):
    aug = (a0_ref[...] + a1_ref[...]) * 0.5
    xt = xt_ref[...]
    xo = xo_ref[...]
    for wref, out, odt in ((wi1_ref, g1_ref, jnp.bfloat16),
                           (wl1_ref, l1_ref, _f32)):
        w = wref[...]
        wo = jnp.dot(wmap_ref[...], w, preferred_element_type=_f32)
        bo = jnp.dot(bmap_ref[...][None, :], w, preferred_element_type=_f32)
        out[0:N_T, :] = (jnp.dot(xt, w[0:D_BASE, :],
                                 preferred_element_type=_f32)
                         + jnp.dot(aug, w[D_BASE:, :],
                                   preferred_element_type=_f32)).astype(odt)
        out[N_T:N, :] = (jnp.dot(xo, wo, preferred_element_type=_f32)
                         + bo).astype(odt)
        out[N:, :] = jnp.zeros((NROWS - N, HID), odt)
    z = jnp.zeros((CNT_STRIDE - R, HID), _f32)
    rp1 = jnp.dot(rel1_ref[...], wi1_ref[...], preferred_element_type=_f32)
    rp1_ref[...] = jnp.concatenate([rp1, z], axis=0)
    rel2 = jnp.dot(rel1_ref[...], wr1_ref[...], preferred_element_type=_f32)
    rp2 = jnp.dot(rel2, wi2_ref[...], preferred_element_type=_f32)
    rp2_ref[...] = jnp.concatenate([rp2, jnp.zeros((CNT_STRIDE - R, OUT), _f32)],
                                   axis=0)
    src2 = ei_ref[0]
    dst2 = ei_ref[1]
    srcp_ref[0:EROWS, :] = src2
    dstp_ref[0:EROWS, :] = dst2
    cidxp_ref[0:EROWS, :] = dst2 * CNT_STRIDE + et_ref[...]
    flat = (lax.broadcasted_iota(jnp.int32, (PROWS, 128), 0) * 128
            + lax.broadcasted_iota(jnp.int32, (PROWS, 128), 1))
    junk = N + lax.rem(flat, NROWS - N)
    srcp_ref[EROWS:, :] = jnp.zeros((PROWS, 128), jnp.int32)
    dstp_ref[EROWS:, :] = junk
    cidxp_ref[EROWS:, :] = junk * CNT_STRIDE
    z1_ref[...] = jnp.zeros((NROWS, HID), jnp.bfloat16)
    zc_ref[...] = jnp.zeros((NROWS * CNT_STRIDE // 128, 128), _f32)
    z2_ref[...] = jnp.zeros((NROWS, OUT), jnp.bfloat16)


def _stage_a(x_target, x_other, a0, a1, W_map, b_map, rel1, W_in1, W_loop1,
             W_rel1, W_in2, ei3, et2):
    return pl.pallas_call(
        _a_body,
        out_shape=(
            jax.ShapeDtypeStruct((NROWS, HID), jnp.bfloat16),
            jax.ShapeDtypeStruct((NROWS, HID), _f32),
            jax.ShapeDtypeStruct((CNT_STRIDE, HID), _f32),
            jax.ShapeDtypeStruct((CNT_STRIDE, OUT), _f32),
            jax.ShapeDtypeStruct((EROWS + PROWS, 128), jnp.int32),
            jax.ShapeDtypeStruct((EROWS + PROWS, 128), jnp.int32),
            jax.ShapeDtypeStruct((EROWS + PROWS, 128), jnp.int32),
            jax.ShapeDtypeStruct((NROWS, HID), jnp.bfloat16),
            jax.ShapeDtypeStruct((NROWS * CNT_STRIDE // 128, 128), _f32),
            jax.ShapeDtypeStruct((NROWS, OUT), jnp.bfloat16),
        ),
    )(x_target, x_other, a0, a1, W_map, b_map, rel1, W_in1, W_loop1,
      W_rel1, W_in2, ei3, et2)


def _seg_kernel_body(width, group, phases, src_hbm, dst_hbm, cidx_hbm,
                     tab_hbm, zrow_hbm, zcnt_hbm, out_rows, out_cnt,
                     src_v, dst_v, cidx_v, buf0, buf1, ones_v, tab_sp,
                     acc, acc_cnt, sem_g0, sem_g1, sem_s0, sem_s1):
    cid = lax.axis_index("c")
    sid = lax.axis_index("s")
    wid = cid * NS + sid
    rpt = NROWS // NS
    pltpu.sync_copy(zrow_hbm.at[pl.ds(sid * rpt, rpt)],
                    acc.at[pl.ds(sid * rpt, rpt)])
    pltpu.sync_copy(tab_hbm.at[pl.ds(sid * rpt, rpt)],
                    tab_sp.at[pl.ds(sid * rpt, rpt)])
    do_cnt = cidx_hbm is not None
    if do_cnt:
        cpt = (NROWS * CNT_STRIDE) // NS
        pltpu.sync_copy(zcnt_hbm.at[pl.ds(sid * cpt, cpt)],
                        acc_cnt.at[pl.ds(sid * cpt, cpt)])
        for i in range(CHUNK // 16):
            ones_v[pl.ds(i * 16, 16)] = jnp.full((16,), 1.0, _f32)
    plsc.subcore_barrier()

    cpp = CHUNKS // phases
    ng = cpp // group

    def fire_gathers(g, buf, sem):
        for b in range(group):
            c = g * group + b
            pltpu.async_copy(tab_sp.at[src_v.at[c]],
                             buf.at[pl.ds(b * CHUNK, CHUNK)], sem)

    def wait_gathers(buf, sem):
        for b in range(group):
            pltpu.make_async_copy(tab_sp.at[src_v.at[0]],
                                  buf.at[pl.ds(b * CHUNK, CHUNK)], sem).wait()

    def fire_scatters(g, buf, sem):
        for b in range(group):
            c = g * group + b
            pltpu.async_copy(buf.at[pl.ds(b * CHUNK, CHUNK)],
                             acc.at[dst_v.at[c]], sem, add=True)
            if do_cnt:
                pltpu.async_copy(ones_v, acc_cnt.at[cidx_v.at[c]], sem,
                                 add=True)

    def wait_scatters(buf, sem):
        for b in range(group):
            pltpu.make_async_copy(buf.at[pl.ds(b * CHUNK, CHUNK)],
                                  acc.at[dst_v.at[0]], sem).wait()
            if do_cnt:
                pltpu.make_async_copy(ones_v, acc_cnt.at[cidx_v.at[0]],
                                      sem).wait()

    for p in range(phases):
        pltpu.sync_copy(src_hbm.at[wid, pl.ds(p * cpp, cpp)], src_v)
        pltpu.sync_copy(dst_hbm.at[wid, pl.ds(p * cpp, cpp)], dst_v)
        if do_cnt:
            pltpu.sync_copy(cidx_hbm.at[wid, pl.ds(p * cpp, cpp)], cidx_v)

        fire_gathers(0, buf0, sem_g0)
        fire_gathers(1, buf1, sem_g1)

        def body(i, carry):
            g = i * 2
            wait_gathers(buf0, sem_g0)
            fire_scatters(g, buf0, sem_s0)
            wait_gathers(buf1, sem_g1)
            wait_scatters(buf0, sem_s0)

            @pl.when(g + 2 < ng)
            def _():
                fire_gathers(g + 2, buf0, sem_g0)

            fire_scatters(g + 1, buf1, sem_s1)
            wait_scatters(buf1, sem_s1)

            @pl.when(g + 3 < ng)
            def _():
                fire_gathers(g + 3, buf1, sem_g1)

            return carry

        lax.fori_loop(0, ng // 2, body, 0)

    plsc.subcore_barrier()
    out_tiles = out_rows.shape[1] // rpt

    @pl.when(sid < out_tiles)
    def _():
        pltpu.sync_copy(acc.at[pl.ds(sid * rpt, rpt)],
                        out_rows.at[cid, pl.ds(sid * rpt, rpt)])
    if do_cnt:
        pltpu.sync_copy(acc_cnt.at[pl.ds(sid * cpt, cpt)],
                        out_cnt.at[cid, pl.ds(sid * cpt, cpt)])


def _sc_layer1(srcp, dstp, cidxp, g1, zrow, zcnt):
    phases = 1
    cpp = CHUNKS // phases

    def body(src_hbm, dst_hbm, cidx_hbm, tab_hbm, zrow_hbm, zcnt_hbm,
             out_rows, out_cnt, src_v, dst_v, cidx_v, buf0, buf1, ones_v,
             tab_sp, acc, acc_cnt, sem_g0, sem_g1, sem_s0, sem_s1):
        _seg_kernel_body(HID, GROUP1, phases, src_hbm, dst_hbm, cidx_hbm,
                         tab_hbm, zrow_hbm, zcnt_hbm, out_rows, out_cnt,
                         src_v, dst_v, cidx_v, buf0, buf1, ones_v, tab_sp,
                         acc, acc_cnt, sem_g0, sem_g1, sem_s0, sem_s1)

    k = pl.kernel(
        body,
        out_type=(
            jax.ShapeDtypeStruct((NC, NROWS, HID), jnp.bfloat16),
            jax.ShapeDtypeStruct((NC, NROWS * CNT_STRIDE), _f32),
        ),
        mesh=plsc.VectorSubcoreMesh(core_axis_name="c", subcore_axis_name="s"),
        compiler_params=pltpu.CompilerParams(use_tc_tiling_on_sc=False),
        scratch_types=[
            pltpu.VMEM((CHUNKS, CHUNK), jnp.int32),
            pltpu.VMEM((CHUNKS, CHUNK), jnp.int32),
            pltpu.VMEM((CHUNKS, CHUNK), jnp.int32),
            pltpu.VMEM((GROUP1 * CHUNK, HID), jnp.bfloat16),
            pltpu.VMEM((GROUP1 * CHUNK, HID), jnp.bfloat16),
            pltpu.VMEM((CHUNK,), _f32),
            pltpu.VMEM_SHARED((NROWS, HID), jnp.bfloat16),
            pltpu.VMEM_SHARED((NROWS, HID), jnp.bfloat16),
            pltpu.VMEM_SHARED((NROWS * CNT_STRIDE,), _f32),
            pltpu.SemaphoreType.DMA,
            pltpu.SemaphoreType.DMA,
            pltpu.SemaphoreType.DMA,
            pltpu.SemaphoreType.DMA,
        ],
    )

    return k(srcp, dstp, cidxp, g1, zrow, zcnt)


def _sc_layer2(srcp, dstp, g2, zrow):
    def body(src_hbm, dst_hbm, tab_hbm, zrow_hbm, out_rows, src_v, dst_v,
             buf0, buf1, tab_sp, acc, sem_g0, sem_g1, sem_s0, sem_s1):
        _seg_kernel_body(OUT, GROUP2, 1, src_hbm, dst_hbm, None, tab_hbm,
                         zrow_hbm, None, out_rows, None, src_v, dst_v, None,
                         buf0, buf1, None, tab_sp, acc, None,
                         sem_g0, sem_g1, sem_s0, sem_s1)

    k = pl.kernel(
        body,
        out_type=jax.ShapeDtypeStruct((NC, 10 * NROWS // NS, OUT),
                                      jnp.bfloat16),
        mesh=plsc.VectorSubcoreMesh(core_axis_name="c", subcore_axis_name="s"),
        compiler_params=pltpu.CompilerParams(use_tc_tiling_on_sc=False),
        scratch_types=[
            pltpu.VMEM((CHUNKS, CHUNK), jnp.int32),
            pltpu.VMEM((CHUNKS, CHUNK), jnp.int32),
            pltpu.VMEM((GROUP2 * CHUNK, OUT), jnp.bfloat16),
            pltpu.VMEM((GROUP2 * CHUNK, OUT), jnp.bfloat16),
            pltpu.VMEM_SHARED((NROWS, OUT), jnp.bfloat16),
            pltpu.VMEM_SHARED((NROWS, OUT), jnp.bfloat16),
            pltpu.SemaphoreType.DMA,
            pltpu.SemaphoreType.DMA,
            pltpu.SemaphoreType.DMA,
            pltpu.SemaphoreType.DMA,
        ],
    )
    return k(srcp, dstp, g2, zrow)


def _b0_body(cntp_ref, rp1_ref, rp2_ref, cc1_ref, cc2_ref):
    cnt = cntp_ref[0] + cntp_ref[1]
    bd1 = jnp.tile(rp1_ref[...], (16, 16))
    ii = lax.broadcasted_iota(jnp.int32, (128, 16 * HID), 0)
    jj = lax.broadcasted_iota(jnp.int32, (128, 16 * HID), 1)
    bd1 = jnp.where(ii // CNT_STRIDE == jj // HID, bd1, 0.0)
    cc1_ref[...] = jnp.dot(cnt, bd1, preferred_element_type=_f32)
    bd2 = jnp.tile(rp2_ref[...], (16, 16))
    i2 = lax.broadcasted_iota(jnp.int32, (128, 16 * OUT), 0)
    j2 = lax.broadcasted_iota(jnp.int32, (128, 16 * OUT), 1)
    bd2 = jnp.where(i2 // CNT_STRIDE == j2 // OUT, bd2, 0.0)
    cc2_ref[...] = jnp.dot(cnt, bd2, preferred_element_type=_f32)


def _stage_b0(cntp3, relp1, relp2):
    return pl.pallas_call(
        _b0_body,
        out_shape=(
            jax.ShapeDtypeStruct((NROWS * CNT_STRIDE // 128, 16 * HID), _f32),
            jax.ShapeDtypeStruct((NROWS * CNT_STRIDE // 128, 16 * OUT), _f32),
        ),
    )(cntp3, relp1, relp2)


def _b1_body(p1_ref, l1_ref, cc1_ref, wi2_ref, wl2_ref, cc2_ref,
             g2_ref, corr2_ref):
    h1 = jnp.maximum(p1_ref[0].astype(_f32) + p1_ref[1].astype(_f32)
                     + l1_ref[...] - cc1_ref[...], 0.0)
    g2_ref[...] = jnp.dot(h1, wi2_ref[...],
                          preferred_element_type=_f32).astype(jnp.bfloat16)
    corr2_ref[...] = (jnp.dot(h1, wl2_ref[...], preferred_element_type=_f32)
                      - cc2_ref[...])


def _stage_b1(p1, loop1, cc1, W_in2, W_loop2, cc2):
    return pl.pallas_call(
        _b1_body,
        out_shape=(
            jax.ShapeDtypeStruct((NROWS, OUT), jnp.bfloat16),
            jax.ShapeDtypeStruct((NROWS, OUT), _f32),
        ),
    )(p1, loop1, cc1, W_in2, W_loop2, cc2)


def _c_body(p2_ref, corr2_ref, out_ref):
    out_ref[...] = (p2_ref[0, :N_T, :].astype(_f32)
                    + p2_ref[1, :N_T, :].astype(_f32)
                    + corr2_ref[:N_T, :])


def _stage_c(p2, corr2):
    return pl.pallas_call(
        _c_body,
        out_shape=jax.ShapeDtypeStruct((N_T, OUT), _f32),
    )(p2, corr2)


def kernel(x_target, x_other, aug_feat_0, aug_feat_1, W_map, b_map, rel1,
           W_in1, W_loop1, W_rel1, W_in2, W_loop2, edge_index, edge_type):
    ei3 = edge_index.astype(jnp.int32).reshape(2, EROWS, 128)
    et2 = edge_type.astype(jnp.int32).reshape(EROWS, 128)

    (g1, loop1, relp1, relp2, srcp2, dstp2, cidxp2, z1, zc, z2) = _stage_a(
        x_target, x_other, aug_feat_0, aug_feat_1, W_map, b_map, rel1,
        W_in1, W_loop1, W_rel1, W_in2, ei3, et2)

    srcp = srcp2.reshape(NW, CHUNKS, CHUNK)
    dstp = dstp2.reshape(NW, CHUNKS, CHUNK)
    cidxp = cidxp2.reshape(NW, CHUNKS, CHUNK)

    p1, cntp = _sc_layer1(srcp, dstp, cidxp, g1, z1,
                          zc.reshape(NROWS * CNT_STRIDE))

    cc1_2d, cc2_2d = _stage_b0(
        cntp.reshape(NC, NROWS * CNT_STRIDE // 128, 128), relp1, relp2)
    g2, corr2 = _stage_b1(p1, loop1, cc1_2d.reshape(NROWS, HID), W_in2,
                          W_loop2, cc2_2d.reshape(NROWS, OUT))

    p2 = _sc_layer2(srcp, dstp, g2, z2)

    return _stage_c(p2, corr2)


# speedup vs baseline: 1.4118x; 1.0195x over previous
import functools

import jax
import jax.numpy as jnp
from jax import lax
from jax.experimental import pallas as pl
from jax.experimental.pallas import tpu as pltpu
from jax.experimental.pallas import tpu_sc as plsc

N_T = 6000
N_O = 4000
N = N_T + N_O
NROWS = 10240
E = 320000
D_BASE = 128
D_OTHER = 256
EMB = 64
IN_DIM = D_BASE + EMB
HID = 64
OUT = 16
R = 4
CNT_STRIDE = 8

NC, NS = 2, 16
NW = NC * NS
CHUNK = 128
GROUP1 = 4
GROUP2 = 8
CHUNKS = 80
EPW = CHUNKS * CHUNK
EPAD = NW * EPW

_f32 = jnp.float32


EROWS = E // 128
PROWS = EPAD // 128 - EROWS


def _a_body(xt_ref, xo_ref, a0_ref, a1_ref, wmap_ref, bmap_ref, rel1_ref,
            wi1_ref, wl1_ref, wr1_ref, wi2_ref, ei_ref, et_ref,
            g1_ref, l1_ref, rp1_ref, rp2_ref, srcp_ref, dstp_ref, cidxp_ref):
    aug = (a0_ref[...] + a1_ref[...]) * 0.5
    xt = xt_ref[...]
    xo = xo_ref[...]
    for wref, out, odt in ((wi1_ref, g1_ref, jnp.bfloat16),
                           (wl1_ref, l1_ref, _f32)):
        w = wref[...]
        wo = jnp.dot(wmap_ref[...], w, preferred_element_type=_f32)
        bo = jnp.dot(bmap_ref[...][None, :], w, preferred_element_type=_f32)
        out[0:N_T, :] = (jnp.dot(xt, w[0:D_BASE, :],
                                 preferred_element_type=_f32)
                         + jnp.dot(aug, w[D_BASE:, :],
                                   preferred_element_type=_f32)).astype(odt)
        out[N_T:N, :] = (jnp.dot(xo, wo, preferred_element_type=_f32)
                         + bo).astype(odt)
        out[N:, :] = jnp.zeros((NROWS - N, HID), odt)
    z = jnp.zeros((CNT_STRIDE - R, HID), _f32)
    rp1 = jnp.dot(rel1_ref[...], wi1_ref[...], preferred_element_type=_f32)
    rp1_ref[...] = jnp.concatenate([rp1, z], axis=0)
    rel2 = jnp.dot(rel1_ref[...], wr1_ref[...], preferred_element_type=_f32)
    rp2 = jnp.dot(rel2, wi2_ref[...], preferred_element_type=_f32)
    rp2_ref[...] = jnp.concatenate([rp2, jnp.zeros((CNT_STRIDE - R, OUT), _f32)],
                                   axis=0)
    src2 = ei_ref[0]
    dst2 = ei_ref[1]
    srcp_ref[0:EROWS, :] = src2
    dstp_ref[0:EROWS, :] = dst2
    cidxp_ref[0:EROWS, :] = dst2 * CNT_STRIDE + et_ref[...]
    flat = (lax.broadcasted_iota(jnp.int32, (PROWS, 128), 0) * 128
            + lax.broadcasted_iota(jnp.int32, (PROWS, 128), 1))
    junk = N + lax.rem(flat, NROWS - N)
    srcp_ref[EROWS:, :] = jnp.zeros((PROWS, 128), jnp.int32)
    dstp_ref[EROWS:, :] = junk
    cidxp_ref[EROWS:, :] = junk * CNT_STRIDE


def _stage_a(x_target, x_other, a0, a1, W_map, b_map, rel1, W_in1, W_loop1,
             W_rel1, W_in2, ei3, et2):
    return pl.pallas_call(
        _a_body,
        out_shape=(
            jax.ShapeDtypeStruct((NROWS, HID), jnp.bfloat16),
            jax.ShapeDtypeStruct((NROWS, HID), _f32),
            jax.ShapeDtypeStruct((CNT_STRIDE, HID), _f32),
            jax.ShapeDtypeStruct((CNT_STRIDE, OUT), _f32),
            jax.ShapeDtypeStruct((EROWS + PROWS, 128), jnp.int32),
            jax.ShapeDtypeStruct((EROWS + PROWS, 128), jnp.int32),
            jax.ShapeDtypeStruct((EROWS + PROWS, 128), jnp.int32),
        ),
    )(x_target, x_other, a0, a1, W_map, b_map, rel1, W_in1, W_loop1,
      W_rel1, W_in2, ei3, et2)


def _seg_kernel_body(width, group, phases, src_hbm, dst_hbm, cidx_hbm,
                     tab_hbm, out_rows, out_cnt,
                     src_v, dst_v, cidx_v, buf0, buf1, ones_v, zcnt_v,
                     tab_sp, acc, acc_cnt, sem_g0, sem_g1, sem_s0, sem_s1):
    cid = lax.axis_index("c")
    sid = lax.axis_index("s")
    wid = cid * NS + sid
    rpt = NROWS // NS
    pltpu.sync_copy(tab_hbm.at[pl.ds(sid * rpt, rpt)],
                    tab_sp.at[pl.ds(sid * rpt, rpt)])

    nzr = buf0.shape[0]
    if width >= 32:
        def zrow_body(r, carry):
            for j in range(width // 32):
                buf0[r, pl.ds(j * 32, 32)] = jnp.zeros((32,), buf0.dtype)
            return carry

        lax.fori_loop(0, nzr, zrow_body, 0)
    else:
        def zrow_body(r, carry):
            buf0[pl.ds(r * 2, 2), :] = jnp.zeros((2, width), buf0.dtype)
            return carry

        lax.fori_loop(0, nzr // 2, zrow_body, 0)
    done = 0
    while done < rpt:
        step = min(nzr, rpt - done)
        pltpu.sync_copy(buf0.at[pl.ds(0, step)],
                        acc.at[pl.ds(sid * rpt + done, step)])
        done += step
    do_cnt = cidx_hbm is not None
    if do_cnt:
        cpt = (NROWS * CNT_STRIDE) // NS

        def zcnt_body(i, carry):
            zcnt_v[pl.ds(i * 16, 16)] = jnp.zeros((16,), _f32)
            return carry

        lax.fori_loop(0, zcnt_v.shape[0] // 16, zcnt_body, 0)
        pltpu.sync_copy(zcnt_v, acc_cnt.at[pl.ds(sid * cpt, cpt)])
        for i in range(CHUNK // 16):
            ones_v[pl.ds(i * 16, 16)] = jnp.full((16,), 1.0, _f32)
    plsc.subcore_barrier()

    cpp = CHUNKS // phases
    ng = cpp // group

    def fire_gathers(g, buf, sem):
        for b in range(group):
            c = g * group + b
            pltpu.async_copy(tab_sp.at[src_v.at[c]],
                             buf.at[pl.ds(b * CHUNK, CHUNK)], sem)

    def wait_gathers(buf, sem):
        for b in range(group):
            pltpu.make_async_copy(tab_sp.at[src_v.at[0]],
                                  buf.at[pl.ds(b * CHUNK, CHUNK)], sem).wait()

    def fire_scatters(g, buf, sem):
        for b in range(group):
            c = g * group + b
            pltpu.async_copy(buf.at[pl.ds(b * CHUNK, CHUNK)],
                             acc.at[dst_v.at[c]], sem, add=True)
            if do_cnt:
                pltpu.async_copy(ones_v, acc_cnt.at[cidx_v.at[c]], sem,
                                 add=True)

    def wait_scatters(buf, sem):
        for b in range(group):
            pltpu.make_async_copy(buf.at[pl.ds(b * CHUNK, CHUNK)],
                                  acc.at[dst_v.at[0]], sem).wait()
            if do_cnt:
                pltpu.make_async_copy(ones_v, acc_cnt.at[cidx_v.at[0]],
                                      sem).wait()

    for p in range(phases):
        pltpu.sync_copy(src_hbm.at[wid, pl.ds(p * cpp, cpp)], src_v)
        pltpu.sync_copy(dst_hbm.at[wid, pl.ds(p * cpp, cpp)], dst_v)
        if do_cnt:
            pltpu.sync_copy(cidx_hbm.at[wid, pl.ds(p * cpp, cpp)], cidx_v)

        fire_gathers(0, buf0, sem_g0)
        fire_gathers(1, buf1, sem_g1)

        def body(i, carry):
            g = i * 2
            wait_gathers(buf0, sem_g0)
            fire_scatters(g, buf0, sem_s0)
            wait_gathers(buf1, sem_g1)
            wait_scatters(buf0, sem_s0)

            @pl.when(g + 2 < ng)
            def _():
                fire_gathers(g + 2, buf0, sem_g0)

            fire_scatters(g + 1, buf1, sem_s1)
            wait_scatters(buf1, sem_s1)

            @pl.when(g + 3 < ng)
            def _():
                fire_gathers(g + 3, buf1, sem_g1)

            return carry

        lax.fori_loop(0, ng // 2, body, 0)

    plsc.subcore_barrier()
    out_tiles = out_rows.shape[1] // rpt

    @pl.when(sid < out_tiles)
    def _():
        pltpu.sync_copy(acc.at[pl.ds(sid * rpt, rpt)],
                        out_rows.at[cid, pl.ds(sid * rpt, rpt)])
    if do_cnt:
        pltpu.sync_copy(acc_cnt.at[pl.ds(sid * cpt, cpt)],
                        out_cnt.at[cid, pl.ds(sid * cpt, cpt)])


def _sc_layer1(srcp, dstp, cidxp, g1):
    phases = 1

    def body(src_hbm, dst_hbm, cidx_hbm, tab_hbm,
             out_rows, out_cnt, src_v, dst_v, cidx_v, buf0, buf1, ones_v,
             zcnt_v, tab_sp, acc, acc_cnt, sem_g0, sem_g1, sem_s0, sem_s1):
        _seg_kernel_body(HID, GROUP1, phases, src_hbm, dst_hbm, cidx_hbm,
                         tab_hbm, out_rows, out_cnt,
                         src_v, dst_v, cidx_v, buf0, buf1, ones_v, zcnt_v,
                         tab_sp, acc, acc_cnt, sem_g0, sem_g1, sem_s0, sem_s1)

    k = pl.kernel(
        body,
        out_type=(
            jax.ShapeDtypeStruct((NC, NROWS, HID), jnp.bfloat16),
            jax.ShapeDtypeStruct((NC, NROWS * CNT_STRIDE), _f32),
        ),
        mesh=plsc.VectorSubcoreMesh(core_axis_name="c", subcore_axis_name="s"),
        compiler_params=pltpu.CompilerParams(use_tc_tiling_on_sc=False),
        scratch_types=[
            pltpu.VMEM((CHUNKS, CHUNK), jnp.int32),
            pltpu.VMEM((CHUNKS, CHUNK), jnp.int32),
            pltpu.VMEM((CHUNKS, CHUNK), jnp.int32),
            pltpu.VMEM((GROUP1 * CHUNK, HID), jnp.bfloat16),
            pltpu.VMEM((GROUP1 * CHUNK, HID), jnp.bfloat16),
            pltpu.VMEM((CHUNK,), _f32),
            pltpu.VMEM((NROWS * CNT_STRIDE // NS,), _f32),
            pltpu.VMEM_SHARED((NROWS, HID), jnp.bfloat16),
            pltpu.VMEM_SHARED((NROWS, HID), jnp.bfloat16),
            pltpu.VMEM_SHARED((NROWS * CNT_STRIDE,), _f32),
            pltpu.SemaphoreType.DMA,
            pltpu.SemaphoreType.DMA,
            pltpu.SemaphoreType.DMA,
            pltpu.SemaphoreType.DMA,
        ],
    )

    return k(srcp, dstp, cidxp, g1)


def _sc_layer2(srcp, dstp, g2):
    def body(src_hbm, dst_hbm, tab_hbm, out_rows, src_v, dst_v,
             buf0, buf1, tab_sp, acc, sem_g0, sem_g1, sem_s0, sem_s1):
        _seg_kernel_body(OUT, GROUP2, 1, src_hbm, dst_hbm, None, tab_hbm,
                         out_rows, None, src_v, dst_v, None,
                         buf0, buf1, None, None, tab_sp, acc, None,
                         sem_g0, sem_g1, sem_s0, sem_s1)

    k = pl.kernel(
        body,
        out_type=jax.ShapeDtypeStruct((NC, 10 * NROWS // NS, OUT),
                                      jnp.bfloat16),
        mesh=plsc.VectorSubcoreMesh(core_axis_name="c", subcore_axis_name="s"),
        compiler_params=pltpu.CompilerParams(use_tc_tiling_on_sc=False),
        scratch_types=[
            pltpu.VMEM((CHUNKS, CHUNK), jnp.int32),
            pltpu.VMEM((CHUNKS, CHUNK), jnp.int32),
            pltpu.VMEM((GROUP2 * CHUNK, OUT), jnp.bfloat16),
            pltpu.VMEM((GROUP2 * CHUNK, OUT), jnp.bfloat16),
            pltpu.VMEM_SHARED((NROWS, OUT), jnp.bfloat16),
            pltpu.VMEM_SHARED((NROWS, OUT), jnp.bfloat16),
            pltpu.SemaphoreType.DMA,
            pltpu.SemaphoreType.DMA,
            pltpu.SemaphoreType.DMA,
            pltpu.SemaphoreType.DMA,
        ],
    )
    return k(srcp, dstp, g2)


def _b0_body(cntp_ref, rp1_ref, rp2_ref, cc1_ref, cc2_ref):
    cnt = cntp_ref[0] + cntp_ref[1]
    bd1 = jnp.tile(rp1_ref[...], (16, 16))
    ii = lax.broadcasted_iota(jnp.int32, (128, 16 * HID), 0)
    jj = lax.broadcasted_iota(jnp.int32, (128, 16 * HID), 1)
    bd1 = jnp.where(ii // CNT_STRIDE == jj // HID, bd1, 0.0)
    cc1_ref[...] = jnp.dot(cnt, bd1, preferred_element_type=_f32)
    bd2 = jnp.tile(rp2_ref[...], (16, 16))
    i2 = lax.broadcasted_iota(jnp.int32, (128, 16 * OUT), 0)
    j2 = lax.broadcasted_iota(jnp.int32, (128, 16 * OUT), 1)
    bd2 = jnp.where(i2 // CNT_STRIDE == j2 // OUT, bd2, 0.0)
    cc2_ref[...] = jnp.dot(cnt, bd2, preferred_element_type=_f32)


def _stage_b0(cntp3, relp1, relp2):
    return pl.pallas_call(
        _b0_body,
        out_shape=(
            jax.ShapeDtypeStruct((NROWS * CNT_STRIDE // 128, 16 * HID), _f32),
            jax.ShapeDtypeStruct((NROWS * CNT_STRIDE // 128, 16 * OUT), _f32),
        ),
    )(cntp3, relp1, relp2)


def _b1_body(p1_ref, l1_ref, cc1_ref, wi2_ref, wl2_ref, cc2_ref,
             g2_ref, corr2_ref):
    h1 = jnp.maximum(p1_ref[0].astype(_f32) + p1_ref[1].astype(_f32)
                     + l1_ref[...] - cc1_ref[...], 0.0)
    g2_ref[...] = jnp.dot(h1, wi2_ref[...],
                          preferred_element_type=_f32).astype(jnp.bfloat16)
    corr2_ref[...] = (jnp.dot(h1, wl2_ref[...], preferred_element_type=_f32)
                      - cc2_ref[...])


def _stage_b1(p1, loop1, cc1, W_in2, W_loop2, cc2):
    return pl.pallas_call(
        _b1_body,
        out_shape=(
            jax.ShapeDtypeStruct((NROWS, OUT), jnp.bfloat16),
            jax.ShapeDtypeStruct((NROWS, OUT), _f32),
        ),
    )(p1, loop1, cc1, W_in2, W_loop2, cc2)


def _c_body(p2_ref, corr2_ref, out_ref):
    out_ref[...] = (p2_ref[0, :N_T, :].astype(_f32)
                    + p2_ref[1, :N_T, :].astype(_f32)
                    + corr2_ref[:N_T, :])


def _stage_c(p2, corr2):
    return pl.pallas_call(
        _c_body,
        out_shape=jax.ShapeDtypeStruct((N_T, OUT), _f32),
    )(p2, corr2)


def kernel(x_target, x_other, aug_feat_0, aug_feat_1, W_map, b_map, rel1,
           W_in1, W_loop1, W_rel1, W_in2, W_loop2, edge_index, edge_type):
    ei3 = edge_index.astype(jnp.int32).reshape(2, EROWS, 128)
    et2 = edge_type.astype(jnp.int32).reshape(EROWS, 128)

    (g1, loop1, relp1, relp2, srcp2, dstp2, cidxp2) = _stage_a(
        x_target, x_other, aug_feat_0, aug_feat_1, W_map, b_map, rel1,
        W_in1, W_loop1, W_rel1, W_in2, ei3, et2)

    srcp = srcp2.reshape(NW, CHUNKS, CHUNK)
    dstp = dstp2.reshape(NW, CHUNKS, CHUNK)
    cidxp = cidxp2.reshape(NW, CHUNKS, CHUNK)

    p1, cntp = _sc_layer1(srcp, dstp, cidxp, g1)

    cc1_2d, cc2_2d = _stage_b0(
        cntp.reshape(NC, NROWS * CNT_STRIDE // 128, 128), relp1, relp2)
    g2, corr2 = _stage_b1(p1, loop1, cc1_2d.reshape(NROWS, HID), W_in2,
                          W_loop2, cc2_2d.reshape(NROWS, OUT))

    p2 = _sc_layer2(srcp, dstp, g2)

    return _stage_c(p2, corr2)
